# baseline (speedup 1.0000x reference)
import functools
from typing import Any

import jax
import jax.numpy as jnp
import numpy as np
from jax.experimental import pallas as pl
from jax.experimental.pallas import tpu as pltpu

B, S, D = 1, 2048, 768
G, H = 4, 3
HD = D // (G * H)
NH = G * H
E, K = 8, 2
DFF = 4 * D
EPS = 1e-6

TT = 256
NT = S // TT
FT = 1024
NF = DFF // FT


def _k1_body(x_ref, w_ref, wqkv_ref, q_ref, k_ref, v_ref):
    x = x_ref[...]
    var = jnp.mean(x * x, axis=-1, keepdims=True)
    h1 = w_ref[...][None, :] * (x * jax.lax.rsqrt(var + EPS))
    qkv = jnp.dot(h1, wqkv_ref[...], preferred_element_type=jnp.float32)
    q_ref[...] = qkv[:, :D]
    k_ref[...] = qkv[:, D:D + G * HD]
    v_ref[...] = qkv[:, D + G * HD:]


def _k1(x2d, norm1_w, wqkv):
    return pl.pallas_call(
        _k1_body,
        grid=(NT,),
        in_specs=[
            pl.BlockSpec((TT, D), lambda t: (t, 0)),
            pl.BlockSpec((D,), lambda t: (0,)),
            pl.BlockSpec((D, D + 2 * G * HD), lambda t: (0, 0)),
        ],
        out_specs=[
            pl.BlockSpec((TT, D), lambda t: (t, 0)),
            pl.BlockSpec((TT, G * HD), lambda t: (t, 0)),
            pl.BlockSpec((TT, G * HD), lambda t: (t, 0)),
        ],
        out_shape=[
            jax.ShapeDtypeStruct((S, D), jnp.float32),
            jax.ShapeDtypeStruct((S, G * HD), jnp.float32),
            jax.ShapeDtypeStruct((S, G * HD), jnp.float32),
        ],
    )(x2d, norm1_w, wqkv)


def _rope(t, cos, sin):
    half = t.shape[-1] // 2
    t1 = t[:, :half]
    t2 = t[:, half:]
    return jnp.concatenate([t1 * cos - t2 * sin, t1 * sin + t2 * cos], axis=-1)


def _k2_body(q_ref, k_ref, v_ref, cosq_ref, sinq_ref, cosk_ref, sink_ref, o_ref):
    t = pl.program_id(1)
    q = _rope(q_ref[0], cosq_ref[...], sinq_ref[...]) * (1.0 / np.sqrt(HD))
    k = _rope(k_ref[0], cosk_ref[...], sink_ref[...])
    scores = jax.lax.dot_general(q, k, (((1,), (1,)), ((), ())),
                                 preferred_element_type=jnp.float32)
    rows = t * TT + jax.lax.broadcasted_iota(jnp.int32, (TT, S), 0)
    cols = jax.lax.broadcasted_iota(jnp.int32, (TT, S), 1)
    scores = jnp.where(cols <= rows, scores, -1e9)
    m = jnp.max(scores, axis=-1, keepdims=True)
    p = jnp.exp(scores - m)
    p = p / jnp.sum(p, axis=-1, keepdims=True)
    o_ref[0] = jnp.dot(p, v_ref[0], preferred_element_type=jnp.float32)


def _k2(q, k, v, cos, sin):
    return pl.pallas_call(
        _k2_body,
        grid=(NH, NT),
        in_specs=[
            pl.BlockSpec((1, TT, HD), lambda h, t: (h, t, 0)),
            pl.BlockSpec((1, S, HD), lambda h, t: (h // H, 0, 0)),
            pl.BlockSpec((1, S, HD), lambda h, t: (h // H, 0, 0)),
            pl.BlockSpec((TT, HD // 2), lambda h, t: (t, 0)),
            pl.BlockSpec((TT, HD // 2), lambda h, t: (t, 0)),
            pl.BlockSpec((S, HD // 2), lambda h, t: (0, 0)),
            pl.BlockSpec((S, HD // 2), lambda h, t: (0, 0)),
        ],
        out_specs=pl.BlockSpec((1, TT, HD), lambda h, t: (h, t, 0)),
        out_shape=jax.ShapeDtypeStruct((NH, S, HD), jnp.float32),
    )(q, k, v, cos, sin, cos, sin)


def _k3_body(attn_ref, wo_ref, x_ref, w2_ref, wr_ref, h2_ref, logits_ref):
    a = attn_ref[...]
    proj = jnp.concatenate(
        [jnp.dot(a[:, g * (H * HD):(g + 1) * (H * HD)], wo_ref[g],
                 preferred_element_type=jnp.float32) for g in range(G)],
        axis=-1)
    x2 = x_ref[...] + proj
    var = jnp.mean(x2 * x2, axis=-1, keepdims=True)
    h2 = w2_ref[...][None, :] * (x2 * jax.lax.rsqrt(var + EPS))
    h2_ref[...] = h2
    logits_ref[...] = jnp.dot(h2, wr_ref[...], preferred_element_type=jnp.float32)


def _k3(attn, wo, x2d, norm2_w, wr):
    return pl.pallas_call(
        _k3_body,
        grid=(NT,),
        in_specs=[
            pl.BlockSpec((TT, D), lambda t: (t, 0)),
            pl.BlockSpec((G, H * HD, H * HD), lambda t: (0, 0, 0)),
            pl.BlockSpec((TT, D), lambda t: (t, 0)),
            pl.BlockSpec((D,), lambda t: (0,)),
            pl.BlockSpec((D, E), lambda t: (0, 0)),
        ],
        out_specs=[
            pl.BlockSpec((TT, D), lambda t: (t, 0)),
            pl.BlockSpec((TT, E), lambda t: (t, 0)),
        ],
        out_shape=[
            jax.ShapeDtypeStruct((S, D), jnp.float32),
            jax.ShapeDtypeStruct((S, E), jnp.float32),
        ],
    )(attn, wo, x2d, norm2_w, wr)


def _k4_body(logits_ref, combine_ref, lb_ref):
    logits = logits_ref[...]
    m = jnp.max(logits, axis=-1, keepdims=True)
    ex = jnp.exp(logits - m)
    probs = ex / jnp.sum(ex, axis=-1, keepdims=True)
    lanes = jax.lax.broadcasted_iota(jnp.int32, (S, E), 1)
    m1 = jnp.max(probs, axis=-1, keepdims=True)
    i1 = jnp.min(jnp.where(probs == m1, lanes, E), axis=-1, keepdims=True)
    sel1 = lanes == i1
    p2 = jnp.where(sel1, -1.0, probs)
    m2 = jnp.max(p2, axis=-1, keepdims=True)
    i2 = jnp.min(jnp.where(p2 == m2, lanes, E), axis=-1, keepdims=True)
    sel2 = lanes == i2
    denom = m1 + m2
    combine = (jnp.where(sel1, m1, 0.0) + jnp.where(sel2, m2, 0.0)) / denom
    combine_ref[...] = combine
    f = jnp.sum(sel1.astype(jnp.float32) + sel2.astype(jnp.float32),
                axis=0) / S
    P = jnp.sum(probs, axis=0) / S
    lb_ref[0, 0] = (E / K) * jnp.sum(f * P)


def _k4(logits):
    return pl.pallas_call(
        _k4_body,
        in_specs=[pl.BlockSpec((S, E), lambda: (0, 0))],
        out_specs=[
            pl.BlockSpec((S, E), lambda: (0, 0)),
            pl.BlockSpec(memory_space=pltpu.SMEM),
        ],
        out_shape=[
            jax.ShapeDtypeStruct((S, E), jnp.float32),
            jax.ShapeDtypeStruct((1, 1), jnp.float32),
        ],
    )(logits)


def _gelu(x):
    c = np.sqrt(2.0 / np.pi).astype(np.float32)
    return 0.5 * x * (1.0 + jnp.tanh(c * (x + 0.044715 * x * x * x)))


def _k5_body(h2_ref, w1_ref, w2_ref, combine_ref, out_ref):
    t, e, f = pl.program_id(0), pl.program_id(1), pl.program_id(2)

    @pl.when(jnp.logical_and(e == 0, f == 0))
    def _():
        out_ref[...] = jnp.zeros_like(out_ref)

    x = h2_ref[...]
    h = _gelu(jnp.dot(x, w1_ref[0], preferred_element_type=jnp.float32))
    y = jnp.dot(h, w2_ref[0], preferred_element_type=jnp.float32)
    cb = combine_ref[pl.ds(t * TT, TT), :]
    lanes = jax.lax.broadcasted_iota(jnp.int32, (TT, E), 1)
    c = jnp.sum(jnp.where(lanes == e, cb, 0.0), axis=-1, keepdims=True)
    out_ref[...] += c * y


def _k5(h2, w1, w2, combine):
    return pl.pallas_call(
        _k5_body,
        grid=(NT, E, NF),
        in_specs=[
            pl.BlockSpec((TT, D), lambda t, e, f: (t, 0)),
            pl.BlockSpec((1, D, FT), lambda t, e, f: (e, 0, f)),
            pl.BlockSpec((1, FT, D), lambda t, e, f: (e, f, 0)),
            pl.BlockSpec((S, E), lambda t, e, f: (0, 0)),
        ],
        out_specs=pl.BlockSpec((TT, D), lambda t, e, f: (t, 0)),
        out_shape=jax.ShapeDtypeStruct((S, D), jnp.float32),
    )(h2, w1, w2, combine)


def kernel(x, norm1_w, norm2_w, Wq, Wk, Wv, Wo, Wr, W1, W2):
    x2d = x.reshape(S, D)
    wq_all = Wq.transpose(1, 0, 2).reshape(D, D)
    wk_all = Wk.transpose(1, 0, 2).reshape(D, G * HD)
    wv_all = Wv.transpose(1, 0, 2).reshape(D, G * HD)
    wqkv = jnp.concatenate([wq_all, wk_all, wv_all], axis=1)

    half = HD // 2
    freqs = 1.0 / (10000.0 ** (jnp.arange(half, dtype=jnp.float32) / half))
    ang = jnp.arange(S, dtype=jnp.float32)[:, None] * freqs
    cos, sin = jnp.cos(ang), jnp.sin(ang)

    q, k, v = _k1(x2d, norm1_w, wqkv)
    q3 = q.reshape(S, NH, HD).transpose(1, 0, 2)
    k3 = k.reshape(S, G, HD).transpose(1, 0, 2)
    v3 = v.reshape(S, G, HD).transpose(1, 0, 2)
    attn = _k2(q3, k3, v3, cos, sin)
    attn2d = attn.transpose(1, 0, 2).reshape(S, NH * HD)
    h2, logits = _k3(attn2d, Wo, x2d, norm2_w, Wr)
    combine, lb = _k4(logits)
    out = _k5(h2, W1, W2, combine)
    return out.reshape(B, S, D), lb[0, 0]


# speedup vs baseline: 1.0473x; 1.0473x over previous
import functools
from typing import Any

import jax
import jax.numpy as jnp
import numpy as np
from jax.experimental import pallas as pl
from jax.experimental.pallas import tpu as pltpu

B, S, D = 1, 2048, 768
G, H = 4, 3
HD = D // (G * H)
NH = G * H
E, K = 8, 2
DFF = 4 * D
EPS = 1e-6

TT = 256
NT = S // TT
FT = 1024
NF = DFF // FT


def _k1_body(x_ref, w_ref, wqkv_ref, q_ref, k_ref, v_ref):
    x = x_ref[...]
    var = jnp.mean(x * x, axis=-1, keepdims=True)
    h1 = w_ref[...][None, :] * (x * jax.lax.rsqrt(var + EPS))
    qkv = jnp.dot(h1, wqkv_ref[...], preferred_element_type=jnp.float32)
    q_ref[...] = qkv[:, :D]
    k_ref[...] = qkv[:, D:D + G * HD]
    v_ref[...] = qkv[:, D + G * HD:]


def _k1(x2d, norm1_w, wqkv):
    return pl.pallas_call(
        _k1_body,
        grid=(NT,),
        in_specs=[
            pl.BlockSpec((TT, D), lambda t: (t, 0)),
            pl.BlockSpec((D,), lambda t: (0,)),
            pl.BlockSpec((D, D + 2 * G * HD), lambda t: (0, 0)),
        ],
        out_specs=[
            pl.BlockSpec((TT, D), lambda t: (t, 0)),
            pl.BlockSpec((TT, G * HD), lambda t: (t, 0)),
            pl.BlockSpec((TT, G * HD), lambda t: (t, 0)),
        ],
        out_shape=[
            jax.ShapeDtypeStruct((S, D), jnp.float32),
            jax.ShapeDtypeStruct((S, G * HD), jnp.float32),
            jax.ShapeDtypeStruct((S, G * HD), jnp.float32),
        ],
    )(x2d, norm1_w, wqkv)


def _rope(t, cos, sin):
    half = t.shape[-1] // 2
    t1 = t[:, :half]
    t2 = t[:, half:]
    return jnp.concatenate([t1 * cos - t2 * sin, t1 * sin + t2 * cos], axis=-1)


def _k2_body(q_ref, k_ref, v_ref, cosq_ref, sinq_ref, cosk_ref, sink_ref, o_ref):
    t = pl.program_id(1)
    q = _rope(q_ref[0], cosq_ref[...], sinq_ref[...]) * (1.0 / np.sqrt(HD))
    k = _rope(k_ref[0], cosk_ref[...], sink_ref[...])
    scores = jax.lax.dot_general(q, k, (((1,), (1,)), ((), ())),
                                 preferred_element_type=jnp.float32)
    rows = t * TT + jax.lax.broadcasted_iota(jnp.int32, (TT, S), 0)
    cols = jax.lax.broadcasted_iota(jnp.int32, (TT, S), 1)
    scores = jnp.where(cols <= rows, scores, -1e9)
    m = jnp.max(scores, axis=-1, keepdims=True)
    p = jnp.exp(scores - m)
    p = p / jnp.sum(p, axis=-1, keepdims=True)
    o_ref[0] = jnp.dot(p, v_ref[0], preferred_element_type=jnp.float32)


def _k2(q, k, v, cos, sin):
    return pl.pallas_call(
        _k2_body,
        grid=(NH, NT),
        in_specs=[
            pl.BlockSpec((1, TT, HD), lambda h, t: (h, t, 0)),
            pl.BlockSpec((1, S, HD), lambda h, t: (h // H, 0, 0)),
            pl.BlockSpec((1, S, HD), lambda h, t: (h // H, 0, 0)),
            pl.BlockSpec((TT, HD // 2), lambda h, t: (t, 0)),
            pl.BlockSpec((TT, HD // 2), lambda h, t: (t, 0)),
            pl.BlockSpec((S, HD // 2), lambda h, t: (0, 0)),
            pl.BlockSpec((S, HD // 2), lambda h, t: (0, 0)),
        ],
        out_specs=pl.BlockSpec((1, TT, HD), lambda h, t: (h, t, 0)),
        out_shape=jax.ShapeDtypeStruct((NH, S, HD), jnp.float32),
    )(q, k, v, cos, sin, cos, sin)


def _k3_body(attn_ref, wo_ref, x_ref, w2_ref, wr_ref, h2_ref, logits_ref):
    a = attn_ref[...]
    proj = jnp.concatenate(
        [jnp.dot(a[:, g * (H * HD):(g + 1) * (H * HD)], wo_ref[g],
                 preferred_element_type=jnp.float32) for g in range(G)],
        axis=-1)
    x2 = x_ref[...] + proj
    var = jnp.mean(x2 * x2, axis=-1, keepdims=True)
    h2 = w2_ref[...][None, :] * (x2 * jax.lax.rsqrt(var + EPS))
    h2_ref[...] = h2
    logits_ref[...] = jnp.dot(h2, wr_ref[...], preferred_element_type=jnp.float32)


def _k3(attn, wo, x2d, norm2_w, wr):
    return pl.pallas_call(
        _k3_body,
        grid=(NT,),
        in_specs=[
            pl.BlockSpec((TT, D), lambda t: (t, 0)),
            pl.BlockSpec((G, H * HD, H * HD), lambda t: (0, 0, 0)),
            pl.BlockSpec((TT, D), lambda t: (t, 0)),
            pl.BlockSpec((D,), lambda t: (0,)),
            pl.BlockSpec((D, E), lambda t: (0, 0)),
        ],
        out_specs=[
            pl.BlockSpec((TT, D), lambda t: (t, 0)),
            pl.BlockSpec((TT, E), lambda t: (t, 0)),
        ],
        out_shape=[
            jax.ShapeDtypeStruct((S, D), jnp.float32),
            jax.ShapeDtypeStruct((S, E), jnp.float32),
        ],
    )(attn, wo, x2d, norm2_w, wr)


def _k4_body(logits_ref, combine_ref, lb_ref):
    logits = logits_ref[...]
    m = jnp.max(logits, axis=-1, keepdims=True)
    ex = jnp.exp(logits - m)
    probs = ex / jnp.sum(ex, axis=-1, keepdims=True)
    lanes = jax.lax.broadcasted_iota(jnp.int32, (S, E), 1)
    m1 = jnp.max(probs, axis=-1, keepdims=True)
    i1 = jnp.min(jnp.where(probs == m1, lanes, E), axis=-1, keepdims=True)
    sel1 = lanes == i1
    p2 = jnp.where(sel1, -1.0, probs)
    m2 = jnp.max(p2, axis=-1, keepdims=True)
    i2 = jnp.min(jnp.where(p2 == m2, lanes, E), axis=-1, keepdims=True)
    sel2 = lanes == i2
    denom = m1 + m2
    combine = (jnp.where(sel1, m1, 0.0) + jnp.where(sel2, m2, 0.0)) / denom
    combine_ref[...] = combine
    f = jnp.sum(sel1.astype(jnp.float32) + sel2.astype(jnp.float32),
                axis=0) / S
    P = jnp.sum(probs, axis=0) / S
    lb_ref[0, 0] = (E / K) * jnp.sum(f * P)


def _k4(logits):
    return pl.pallas_call(
        _k4_body,
        in_specs=[pl.BlockSpec((S, E), lambda: (0, 0))],
        out_specs=[
            pl.BlockSpec((S, E), lambda: (0, 0)),
            pl.BlockSpec(memory_space=pltpu.SMEM),
        ],
        out_shape=[
            jax.ShapeDtypeStruct((S, E), jnp.float32),
            jax.ShapeDtypeStruct((1, 1), jnp.float32),
        ],
    )(logits)


def _gelu(x):
    c = np.sqrt(2.0 / np.pi).astype(np.float32)
    return 0.5 * x * (1.0 + jnp.tanh(c * (x + 0.044715 * x * x * x)))


def _k5_body(h2_ref, w1_ref, w2_ref, combine_ref, out_ref):
    t, e, f = pl.program_id(0), pl.program_id(1), pl.program_id(2)

    @pl.when(jnp.logical_and(e == 0, f == 0))
    def _():
        out_ref[...] = jnp.zeros_like(out_ref)

    x = h2_ref[...].astype(jnp.bfloat16)
    h = _gelu(jnp.dot(x, w1_ref[0], preferred_element_type=jnp.float32))
    y = jnp.dot(h.astype(jnp.bfloat16), w2_ref[0], preferred_element_type=jnp.float32)
    cb = combine_ref[pl.ds(t * TT, TT), :]
    lanes = jax.lax.broadcasted_iota(jnp.int32, (TT, E), 1)
    c = jnp.sum(jnp.where(lanes == e, cb, 0.0), axis=-1, keepdims=True)
    out_ref[...] += c * y


def _k5(h2, w1, w2, combine):
    return pl.pallas_call(
        _k5_body,
        grid=(NT, E, NF),
        in_specs=[
            pl.BlockSpec((TT, D), lambda t, e, f: (t, 0)),
            pl.BlockSpec((1, D, FT), lambda t, e, f: (e, 0, f)),
            pl.BlockSpec((1, FT, D), lambda t, e, f: (e, f, 0)),
            pl.BlockSpec((S, E), lambda t, e, f: (0, 0)),
        ],
        out_specs=pl.BlockSpec((TT, D), lambda t, e, f: (t, 0)),
        out_shape=jax.ShapeDtypeStruct((S, D), jnp.float32),
    )(h2, w1.astype(jnp.bfloat16), w2.astype(jnp.bfloat16), combine)


def kernel(x, norm1_w, norm2_w, Wq, Wk, Wv, Wo, Wr, W1, W2):
    x2d = x.reshape(S, D)
    wq_all = Wq.transpose(1, 0, 2).reshape(D, D)
    wk_all = Wk.transpose(1, 0, 2).reshape(D, G * HD)
    wv_all = Wv.transpose(1, 0, 2).reshape(D, G * HD)
    wqkv = jnp.concatenate([wq_all, wk_all, wv_all], axis=1)

    half = HD // 2
    freqs = 1.0 / (10000.0 ** (jnp.arange(half, dtype=jnp.float32) / half))
    ang = jnp.arange(S, dtype=jnp.float32)[:, None] * freqs
    cos, sin = jnp.cos(ang), jnp.sin(ang)

    q, k, v = _k1(x2d, norm1_w, wqkv)
    q3 = q.reshape(S, NH, HD).transpose(1, 0, 2)
    k3 = k.reshape(S, G, HD).transpose(1, 0, 2)
    v3 = v.reshape(S, G, HD).transpose(1, 0, 2)
    attn = _k2(q3, k3, v3, cos, sin)
    attn2d = attn.transpose(1, 0, 2).reshape(S, NH * HD)
    h2, logits = _k3(attn2d, Wo, x2d, norm2_w, Wr)
    combine, lb = _k4(logits)
    out = _k5(h2, W1, W2, combine)
    return out.reshape(B, S, D), lb[0, 0]


# speedup vs baseline: 1.2368x; 1.1810x over previous
import functools
from typing import Any

import jax
import jax.numpy as jnp
import numpy as np
from jax.experimental import pallas as pl
from jax.experimental.pallas import tpu as pltpu

B, S, D = 1, 2048, 768
G, H = 4, 3
HD = D // (G * H)
NH = G * H
E, K = 8, 2
DFF = 4 * D
EPS = 1e-6

TT = 256
NT = S // TT
FT = 1024
NF = DFF // FT


def _k1_body(x_ref, w_ref, wqkv_ref, q_ref, k_ref, v_ref):
    x = x_ref[...]
    var = jnp.mean(x * x, axis=-1, keepdims=True)
    h1 = w_ref[...][None, :] * (x * jax.lax.rsqrt(var + EPS))
    qkv = jnp.dot(h1, wqkv_ref[...], preferred_element_type=jnp.float32)
    q_ref[...] = qkv[:, :D]
    k_ref[...] = qkv[:, D:D + G * HD]
    v_ref[...] = qkv[:, D + G * HD:]


def _k1(x2d, norm1_w, wqkv):
    return pl.pallas_call(
        _k1_body,
        grid=(NT,),
        in_specs=[
            pl.BlockSpec((TT, D), lambda t: (t, 0)),
            pl.BlockSpec((D,), lambda t: (0,)),
            pl.BlockSpec((D, D + 2 * G * HD), lambda t: (0, 0)),
        ],
        out_specs=[
            pl.BlockSpec((TT, D), lambda t: (t, 0)),
            pl.BlockSpec((TT, G * HD), lambda t: (t, 0)),
            pl.BlockSpec((TT, G * HD), lambda t: (t, 0)),
        ],
        out_shape=[
            jax.ShapeDtypeStruct((S, D), jnp.float32),
            jax.ShapeDtypeStruct((S, G * HD), jnp.float32),
            jax.ShapeDtypeStruct((S, G * HD), jnp.float32),
        ],
    )(x2d, norm1_w, wqkv)


def _rope(t, cos, sin):
    half = t.shape[-1] // 2
    t1 = t[:, :half]
    t2 = t[:, half:]
    return jnp.concatenate([t1 * cos - t2 * sin, t1 * sin + t2 * cos], axis=-1)


def _k2_body(q_ref, k_ref, v_ref, cosq_ref, sinq_ref, cosk_ref, sink_ref, o_ref):
    t = pl.program_id(1)
    q = _rope(q_ref[0], cosq_ref[...], sinq_ref[...]) * (1.0 / np.sqrt(HD))
    k = _rope(k_ref[0], cosk_ref[...], sink_ref[...])
    scores = jax.lax.dot_general(q, k, (((1,), (1,)), ((), ())),
                                 preferred_element_type=jnp.float32)
    rows = t * TT + jax.lax.broadcasted_iota(jnp.int32, (TT, S), 0)
    cols = jax.lax.broadcasted_iota(jnp.int32, (TT, S), 1)
    scores = jnp.where(cols <= rows, scores, -1e9)
    m = jnp.max(scores, axis=-1, keepdims=True)
    p = jnp.exp(scores - m)
    p = p / jnp.sum(p, axis=-1, keepdims=True)
    o_ref[0] = jnp.dot(p, v_ref[0], preferred_element_type=jnp.float32)


def _k2(q, k, v, cos, sin):
    return pl.pallas_call(
        _k2_body,
        grid=(NH, NT),
        in_specs=[
            pl.BlockSpec((1, TT, HD), lambda h, t: (h, t, 0)),
            pl.BlockSpec((1, S, HD), lambda h, t: (h // H, 0, 0)),
            pl.BlockSpec((1, S, HD), lambda h, t: (h // H, 0, 0)),
            pl.BlockSpec((TT, HD // 2), lambda h, t: (t, 0)),
            pl.BlockSpec((TT, HD // 2), lambda h, t: (t, 0)),
            pl.BlockSpec((S, HD // 2), lambda h, t: (0, 0)),
            pl.BlockSpec((S, HD // 2), lambda h, t: (0, 0)),
        ],
        out_specs=pl.BlockSpec((1, TT, HD), lambda h, t: (h, t, 0)),
        out_shape=jax.ShapeDtypeStruct((NH, S, HD), jnp.float32),
    )(q, k, v, cos, sin, cos, sin)


def _k3_body(attn_ref, wo_ref, x_ref, w2_ref, wr_ref, h2_ref, logits_ref):
    a = attn_ref[...]
    proj = jnp.concatenate(
        [jnp.dot(a[:, g * (H * HD):(g + 1) * (H * HD)], wo_ref[g],
                 preferred_element_type=jnp.float32) for g in range(G)],
        axis=-1)
    x2 = x_ref[...] + proj
    var = jnp.mean(x2 * x2, axis=-1, keepdims=True)
    h2 = w2_ref[...][None, :] * (x2 * jax.lax.rsqrt(var + EPS))
    h2_ref[...] = h2
    logits_ref[...] = jnp.dot(h2, wr_ref[...], preferred_element_type=jnp.float32)


def _k3(attn, wo, x2d, norm2_w, wr):
    return pl.pallas_call(
        _k3_body,
        grid=(NT,),
        in_specs=[
            pl.BlockSpec((TT, D), lambda t: (t, 0)),
            pl.BlockSpec((G, H * HD, H * HD), lambda t: (0, 0, 0)),
            pl.BlockSpec((TT, D), lambda t: (t, 0)),
            pl.BlockSpec((D,), lambda t: (0,)),
            pl.BlockSpec((D, E), lambda t: (0, 0)),
        ],
        out_specs=[
            pl.BlockSpec((TT, D), lambda t: (t, 0)),
            pl.BlockSpec((TT, E), lambda t: (t, 0)),
        ],
        out_shape=[
            jax.ShapeDtypeStruct((S, D), jnp.float32),
            jax.ShapeDtypeStruct((S, E), jnp.float32),
        ],
    )(attn, wo, x2d, norm2_w, wr)


def _k4_body(logits_ref, combine_ref, lb_ref):
    logits = logits_ref[...]
    m = jnp.max(logits, axis=-1, keepdims=True)
    ex = jnp.exp(logits - m)
    probs = ex / jnp.sum(ex, axis=-1, keepdims=True)
    lanes = jax.lax.broadcasted_iota(jnp.int32, (S, E), 1)
    m1 = jnp.max(probs, axis=-1, keepdims=True)
    i1 = jnp.min(jnp.where(probs == m1, lanes, E), axis=-1, keepdims=True)
    sel1 = lanes == i1
    p2 = jnp.where(sel1, -1.0, probs)
    m2 = jnp.max(p2, axis=-1, keepdims=True)
    i2 = jnp.min(jnp.where(p2 == m2, lanes, E), axis=-1, keepdims=True)
    sel2 = lanes == i2
    denom = m1 + m2
    combine = (jnp.where(sel1, m1, 0.0) + jnp.where(sel2, m2, 0.0)) / denom
    combine_ref[...] = combine
    f = jnp.sum(sel1.astype(jnp.float32) + sel2.astype(jnp.float32),
                axis=0) / S
    P = jnp.sum(probs, axis=0) / S
    lb_ref[0, 0] = (E / K) * jnp.sum(f * P)


def _k4(logits):
    return pl.pallas_call(
        _k4_body,
        in_specs=[pl.BlockSpec((S, E), lambda: (0, 0))],
        out_specs=[
            pl.BlockSpec((S, E), lambda: (0, 0)),
            pl.BlockSpec(memory_space=pltpu.SMEM),
        ],
        out_shape=[
            jax.ShapeDtypeStruct((S, E), jnp.float32),
            jax.ShapeDtypeStruct((1, 1), jnp.float32),
        ],
    )(logits)


def _gelu(x):
    c = np.sqrt(2.0 / np.pi).astype(np.float32)
    return 0.5 * x * (1.0 + jnp.tanh(c * (x + 0.044715 * x * x * x)))


def _k5_body(h2_ref, w1_ref, w2_ref, combine_ref, out_ref):
    e, f = pl.program_id(0), pl.program_id(1)

    @pl.when(jnp.logical_and(e == 0, f == 0))
    def _():
        out_ref[...] = jnp.zeros_like(out_ref)

    x = h2_ref[...].astype(jnp.bfloat16)
    h = _gelu(jnp.dot(x, w1_ref[0], preferred_element_type=jnp.float32))
    y = jnp.dot(h.astype(jnp.bfloat16), w2_ref[0], preferred_element_type=jnp.float32)
    cb = combine_ref[...]
    lanes = jax.lax.broadcasted_iota(jnp.int32, (S, E), 1)
    c = jnp.sum(jnp.where(lanes == e, cb, 0.0), axis=-1, keepdims=True)
    out_ref[...] += c * y


def _k5(h2, w1, w2, combine):
    return pl.pallas_call(
        _k5_body,
        grid=(E, NF),
        in_specs=[
            pl.BlockSpec((S, D), lambda e, f: (0, 0)),
            pl.BlockSpec((1, D, FT), lambda e, f: (e, 0, f)),
            pl.BlockSpec((1, FT, D), lambda e, f: (e, f, 0)),
            pl.BlockSpec((S, E), lambda e, f: (0, 0)),
        ],
        out_specs=pl.BlockSpec((S, D), lambda e, f: (0, 0)),
        out_shape=jax.ShapeDtypeStruct((S, D), jnp.float32),
    )(h2, w1.astype(jnp.bfloat16), w2.astype(jnp.bfloat16), combine)


def kernel(x, norm1_w, norm2_w, Wq, Wk, Wv, Wo, Wr, W1, W2):
    x2d = x.reshape(S, D)
    wq_all = Wq.transpose(1, 0, 2).reshape(D, D)
    wk_all = Wk.transpose(1, 0, 2).reshape(D, G * HD)
    wv_all = Wv.transpose(1, 0, 2).reshape(D, G * HD)
    wqkv = jnp.concatenate([wq_all, wk_all, wv_all], axis=1)

    half = HD // 2
    freqs = 1.0 / (10000.0 ** (jnp.arange(half, dtype=jnp.float32) / half))
    ang = jnp.arange(S, dtype=jnp.float32)[:, None] * freqs
    cos, sin = jnp.cos(ang), jnp.sin(ang)

    q, k, v = _k1(x2d, norm1_w, wqkv)
    q3 = q.reshape(S, NH, HD).transpose(1, 0, 2)
    k3 = k.reshape(S, G, HD).transpose(1, 0, 2)
    v3 = v.reshape(S, G, HD).transpose(1, 0, 2)
    attn = _k2(q3, k3, v3, cos, sin)
    attn2d = attn.transpose(1, 0, 2).reshape(S, NH * HD)
    h2, logits = _k3(attn2d, Wo, x2d, norm2_w, Wr)
    combine, lb = _k4(logits)
    out = _k5(h2, W1, W2, combine)
    return out.reshape(B, S, D), lb[0, 0]


# speedup vs baseline: 1.3945x; 1.1275x over previous
import functools
from typing import Any

import jax
import jax.numpy as jnp
import numpy as np
from jax.experimental import pallas as pl
from jax.experimental.pallas import tpu as pltpu

B, S, D = 1, 2048, 768
G, H = 4, 3
HD = D // (G * H)
NH = G * H
E, K = 8, 2
DFF = 4 * D
EPS = 1e-6

TT = 256
NT = S // TT
FT = 1024
NF = DFF // FT


def _rope_cols(t, nheads, cos, sin):
    pieces = []
    for h in range(nheads):
        a = t[:, h * HD:h * HD + HD // 2]
        b = t[:, h * HD + HD // 2:(h + 1) * HD]
        pieces.append(a * cos - b * sin)
        pieces.append(a * sin + b * cos)
    return jnp.concatenate(pieces, axis=-1)


def _k1_body(x_ref, w_ref, wqkv_ref, cos_ref, sin_ref, q_ref, k_ref, v_ref):
    x = x_ref[...]
    var = jnp.mean(x * x, axis=-1, keepdims=True)
    h1 = w_ref[...][None, :] * (x * jax.lax.rsqrt(var + EPS))
    qkv = jnp.dot(h1, wqkv_ref[...], preferred_element_type=jnp.float32)
    cos, sin = cos_ref[...], sin_ref[...]
    q_ref[...] = (_rope_cols(qkv[:, :D], NH, cos, sin)
                  * (1.0 / np.sqrt(HD))).astype(jnp.bfloat16)
    k_ref[...] = _rope_cols(qkv[:, D:D + G * HD], G, cos, sin).astype(jnp.bfloat16)
    v_ref[...] = qkv[:, D + G * HD:].astype(jnp.bfloat16)


def _k1(x2d, norm1_w, wqkv, cos, sin):
    return pl.pallas_call(
        _k1_body,
        grid=(NT,),
        in_specs=[
            pl.BlockSpec((TT, D), lambda t: (t, 0)),
            pl.BlockSpec((D,), lambda t: (0,)),
            pl.BlockSpec((D, D + 2 * G * HD), lambda t: (0, 0)),
            pl.BlockSpec((TT, HD // 2), lambda t: (t, 0)),
            pl.BlockSpec((TT, HD // 2), lambda t: (t, 0)),
        ],
        out_specs=[
            pl.BlockSpec((TT, D), lambda t: (t, 0)),
            pl.BlockSpec((TT, G * HD), lambda t: (t, 0)),
            pl.BlockSpec((TT, G * HD), lambda t: (t, 0)),
        ],
        out_shape=[
            jax.ShapeDtypeStruct((S, D), jnp.bfloat16),
            jax.ShapeDtypeStruct((S, G * HD), jnp.bfloat16),
            jax.ShapeDtypeStruct((S, G * HD), jnp.bfloat16),
        ],
    )(x2d, norm1_w, wqkv, cos, sin)


def _k2_body(q_ref, kt_ref, v_ref, o_ref):
    t = pl.program_id(1)
    q = q_ref[0]

    def chunk(c, carry):
        acc, m, l = carry
        off = pl.multiple_of(c * TT, TT)
        kt = kt_ref[0, :, pl.ds(off, TT)]
        vc = v_ref[0, pl.ds(off, TT), :]
        sc = jnp.dot(q, kt, preferred_element_type=jnp.float32)
        rows = jax.lax.broadcasted_iota(jnp.int32, (TT, TT), 0)
        cols = jax.lax.broadcasted_iota(jnp.int32, (TT, TT), 1)
        sc = jnp.where(jnp.logical_or(c < t, cols <= rows), sc, -1e9)
        m_new = jnp.maximum(m, jnp.max(sc, axis=-1, keepdims=True))
        p = jnp.exp(sc - m_new)
        alpha = jnp.exp(m - m_new)
        l = l * alpha + jnp.sum(p, axis=-1, keepdims=True)
        pv = jnp.dot(p.astype(jnp.bfloat16), vc,
                     preferred_element_type=jnp.float32)
        return acc * alpha + pv, m_new, l

    acc0 = jnp.zeros((TT, HD), jnp.float32)
    m0 = jnp.full((TT, 1), -1e30, jnp.float32)
    l0 = jnp.zeros((TT, 1), jnp.float32)
    acc, m, l = jax.lax.fori_loop(0, t + 1, chunk, (acc0, m0, l0))
    o_ref[0] = acc / l


def _k2(q, kt, v):
    return pl.pallas_call(
        _k2_body,
        grid=(NH, NT),
        in_specs=[
            pl.BlockSpec((1, TT, HD), lambda h, t: (h, t, 0)),
            pl.BlockSpec((1, HD, S), lambda h, t: (h // H, 0, 0)),
            pl.BlockSpec((1, S, HD), lambda h, t: (h // H, 0, 0)),
        ],
        out_specs=pl.BlockSpec((1, TT, HD), lambda h, t: (h, t, 0)),
        out_shape=jax.ShapeDtypeStruct((NH, S, HD), jnp.float32),
    )(q, kt, v)


def _k3_body(attn_ref, wo_ref, x_ref, w2_ref, wr_ref, h2_ref, logits_ref):
    a = attn_ref[...]
    proj = jnp.concatenate(
        [jnp.dot(a[:, g * (H * HD):(g + 1) * (H * HD)], wo_ref[g],
                 preferred_element_type=jnp.float32) for g in range(G)],
        axis=-1)
    x2 = x_ref[...] + proj
    var = jnp.mean(x2 * x2, axis=-1, keepdims=True)
    h2 = w2_ref[...][None, :] * (x2 * jax.lax.rsqrt(var + EPS))
    h2_ref[...] = h2
    logits_ref[...] = jnp.dot(h2, wr_ref[...], preferred_element_type=jnp.float32)


def _k3(attn, wo, x2d, norm2_w, wr):
    return pl.pallas_call(
        _k3_body,
        grid=(NT,),
        in_specs=[
            pl.BlockSpec((TT, D), lambda t: (t, 0)),
            pl.BlockSpec((G, H * HD, H * HD), lambda t: (0, 0, 0)),
            pl.BlockSpec((TT, D), lambda t: (t, 0)),
            pl.BlockSpec((D,), lambda t: (0,)),
            pl.BlockSpec((D, E), lambda t: (0, 0)),
        ],
        out_specs=[
            pl.BlockSpec((TT, D), lambda t: (t, 0)),
            pl.BlockSpec((TT, E), lambda t: (t, 0)),
        ],
        out_shape=[
            jax.ShapeDtypeStruct((S, D), jnp.float32),
            jax.ShapeDtypeStruct((S, E), jnp.float32),
        ],
    )(attn, wo, x2d, norm2_w, wr)


def _k4_body(logits_ref, combine_ref, lb_ref):
    logits = logits_ref[...]
    m = jnp.max(logits, axis=-1, keepdims=True)
    ex = jnp.exp(logits - m)
    probs = ex / jnp.sum(ex, axis=-1, keepdims=True)
    lanes = jax.lax.broadcasted_iota(jnp.int32, (S, E), 1)
    m1 = jnp.max(probs, axis=-1, keepdims=True)
    i1 = jnp.min(jnp.where(probs == m1, lanes, E), axis=-1, keepdims=True)
    sel1 = lanes == i1
    p2 = jnp.where(sel1, -1.0, probs)
    m2 = jnp.max(p2, axis=-1, keepdims=True)
    i2 = jnp.min(jnp.where(p2 == m2, lanes, E), axis=-1, keepdims=True)
    sel2 = lanes == i2
    denom = m1 + m2
    combine = (jnp.where(sel1, m1, 0.0) + jnp.where(sel2, m2, 0.0)) / denom
    combine_ref[...] = combine
    f = jnp.sum(sel1.astype(jnp.float32) + sel2.astype(jnp.float32),
                axis=0) / S
    P = jnp.sum(probs, axis=0) / S
    lb_ref[0, 0] = (E / K) * jnp.sum(f * P)


def _k4(logits):
    return pl.pallas_call(
        _k4_body,
        in_specs=[pl.BlockSpec((S, E), lambda: (0, 0))],
        out_specs=[
            pl.BlockSpec((S, E), lambda: (0, 0)),
            pl.BlockSpec(memory_space=pltpu.SMEM),
        ],
        out_shape=[
            jax.ShapeDtypeStruct((S, E), jnp.float32),
            jax.ShapeDtypeStruct((1, 1), jnp.float32),
        ],
    )(logits)


def _gelu(x):
    c = np.sqrt(2.0 / np.pi).astype(np.float32)
    return 0.5 * x * (1.0 + jnp.tanh(c * (x + 0.044715 * x * x * x)))


def _k5_body(h2_ref, w1_ref, w2_ref, combine_ref, out_ref):
    e, f = pl.program_id(0), pl.program_id(1)

    @pl.when(jnp.logical_and(e == 0, f == 0))
    def _():
        out_ref[...] = jnp.zeros_like(out_ref)

    x = h2_ref[...].astype(jnp.bfloat16)
    h = _gelu(jnp.dot(x, w1_ref[0], preferred_element_type=jnp.float32))
    y = jnp.dot(h.astype(jnp.bfloat16), w2_ref[0], preferred_element_type=jnp.float32)
    cb = combine_ref[...]
    lanes = jax.lax.broadcasted_iota(jnp.int32, (S, E), 1)
    c = jnp.sum(jnp.where(lanes == e, cb, 0.0), axis=-1, keepdims=True)
    out_ref[...] += c * y


def _k5(h2, w1, w2, combine):
    return pl.pallas_call(
        _k5_body,
        grid=(E, NF),
        in_specs=[
            pl.BlockSpec((S, D), lambda e, f: (0, 0)),
            pl.BlockSpec((1, D, FT), lambda e, f: (e, 0, f)),
            pl.BlockSpec((1, FT, D), lambda e, f: (e, f, 0)),
            pl.BlockSpec((S, E), lambda e, f: (0, 0)),
        ],
        out_specs=pl.BlockSpec((S, D), lambda e, f: (0, 0)),
        out_shape=jax.ShapeDtypeStruct((S, D), jnp.float32),
    )(h2, w1.astype(jnp.bfloat16), w2.astype(jnp.bfloat16), combine)


def kernel(x, norm1_w, norm2_w, Wq, Wk, Wv, Wo, Wr, W1, W2):
    x2d = x.reshape(S, D)
    wq_all = Wq.transpose(1, 0, 2).reshape(D, D)
    wk_all = Wk.transpose(1, 0, 2).reshape(D, G * HD)
    wv_all = Wv.transpose(1, 0, 2).reshape(D, G * HD)
    wqkv = jnp.concatenate([wq_all, wk_all, wv_all], axis=1)

    half = HD // 2
    freqs = 1.0 / (10000.0 ** (jnp.arange(half, dtype=jnp.float32) / half))
    ang = jnp.arange(S, dtype=jnp.float32)[:, None] * freqs
    cos, sin = jnp.cos(ang), jnp.sin(ang)

    q, k, v = _k1(x2d, norm1_w, wqkv, cos, sin)
    q3 = q.reshape(S, NH, HD).transpose(1, 0, 2)
    kt = k.reshape(S, G, HD).transpose(1, 2, 0)
    v3 = v.reshape(S, G, HD).transpose(1, 0, 2)
    attn = _k2(q3, kt, v3)
    attn2d = attn.transpose(1, 0, 2).reshape(S, NH * HD)
    h2, logits = _k3(attn2d, Wo, x2d, norm2_w, Wr)
    combine, lb = _k4(logits)
    out = _k5(h2, W1, W2, combine)
    return out.reshape(B, S, D), lb[0, 0]


# speedup vs baseline: 1.6303x; 1.1691x over previous
import functools
from typing import Any

import jax
import jax.numpy as jnp
import numpy as np
from jax import lax
from jax.experimental import pallas as pl
from jax.experimental.pallas import tpu as pltpu
from jax.experimental.pallas import tpu_sc as plsc

B, S, D = 1, 2048, 768
G, H = 4, 3
HD = D // (G * H)
NH = G * H
E, K = 8, 2
DFF = 4 * D
EPS = 1e-6

TT = 256
NT = S // TT
FT = 1024
NF = DFF // FT

BM = 256
NTILES = (K * S) // BM + E
GPAD = NTILES * BM
CPW = S // 32


def _rope_cols(t, nheads, cos, sin):
    pieces = []
    for h in range(nheads):
        a = t[:, h * HD:h * HD + HD // 2]
        b = t[:, h * HD + HD // 2:(h + 1) * HD]
        pieces.append(a * cos - b * sin)
        pieces.append(a * sin + b * cos)
    return jnp.concatenate(pieces, axis=-1)


def _k1_body(x_ref, w_ref, wqkv_ref, cos_ref, sin_ref, q_ref, k_ref, v_ref):
    x = x_ref[...]
    var = jnp.mean(x * x, axis=-1, keepdims=True)
    h1 = w_ref[...][None, :] * (x * jax.lax.rsqrt(var + EPS))
    qkv = jnp.dot(h1, wqkv_ref[...], preferred_element_type=jnp.float32)
    cos, sin = cos_ref[...], sin_ref[...]
    q_ref[...] = (_rope_cols(qkv[:, :D], NH, cos, sin)
                  * (1.0 / np.sqrt(HD))).astype(jnp.bfloat16)
    k_ref[...] = _rope_cols(qkv[:, D:D + G * HD], G, cos, sin).astype(jnp.bfloat16)
    v_ref[...] = qkv[:, D + G * HD:].astype(jnp.bfloat16)


def _k1(x2d, norm1_w, wqkv, cos, sin):
    return pl.pallas_call(
        _k1_body,
        grid=(NT,),
        in_specs=[
            pl.BlockSpec((TT, D), lambda t: (t, 0)),
            pl.BlockSpec((D,), lambda t: (0,)),
            pl.BlockSpec((D, D + 2 * G * HD), lambda t: (0, 0)),
            pl.BlockSpec((TT, HD // 2), lambda t: (t, 0)),
            pl.BlockSpec((TT, HD // 2), lambda t: (t, 0)),
        ],
        out_specs=[
            pl.BlockSpec((TT, D), lambda t: (t, 0)),
            pl.BlockSpec((TT, G * HD), lambda t: (t, 0)),
            pl.BlockSpec((TT, G * HD), lambda t: (t, 0)),
        ],
        out_shape=[
            jax.ShapeDtypeStruct((S, D), jnp.bfloat16),
            jax.ShapeDtypeStruct((S, G * HD), jnp.bfloat16),
            jax.ShapeDtypeStruct((S, G * HD), jnp.bfloat16),
        ],
    )(x2d, norm1_w, wqkv, cos, sin)


def _k2_body(q_ref, kt_ref, v_ref, o_ref):
    t = pl.program_id(1)
    q = q_ref[0]

    def chunk(c, carry):
        acc, m, l = carry
        off = pl.multiple_of(c * TT, TT)
        kt = kt_ref[0, :, pl.ds(off, TT)]
        vc = v_ref[0, pl.ds(off, TT), :]
        sc = jnp.dot(q, kt, preferred_element_type=jnp.float32)
        rows = jax.lax.broadcasted_iota(jnp.int32, (TT, TT), 0)
        cols = jax.lax.broadcasted_iota(jnp.int32, (TT, TT), 1)
        sc = jnp.where(jnp.logical_or(c < t, cols <= rows), sc, -1e9)
        m_new = jnp.maximum(m, jnp.max(sc, axis=-1, keepdims=True))
        p = jnp.exp(sc - m_new)
        alpha = jnp.exp(m - m_new)
        l = l * alpha + jnp.sum(p, axis=-1, keepdims=True)
        pv = jnp.dot(p.astype(jnp.bfloat16), vc,
                     preferred_element_type=jnp.float32)
        return acc * alpha + pv, m_new, l

    acc0 = jnp.zeros((TT, HD), jnp.float32)
    m0 = jnp.full((TT, 1), -1e30, jnp.float32)
    l0 = jnp.zeros((TT, 1), jnp.float32)
    acc, m, l = jax.lax.fori_loop(0, t + 1, chunk, (acc0, m0, l0))
    o_ref[0] = acc / l


def _k2(q, kt, v):
    return pl.pallas_call(
        _k2_body,
        grid=(NH, NT),
        in_specs=[
            pl.BlockSpec((1, TT, HD), lambda h, t: (h, t, 0)),
            pl.BlockSpec((1, HD, S), lambda h, t: (h // H, 0, 0)),
            pl.BlockSpec((1, S, HD), lambda h, t: (h // H, 0, 0)),
        ],
        out_specs=pl.BlockSpec((1, TT, HD), lambda h, t: (h, t, 0)),
        out_shape=jax.ShapeDtypeStruct((NH, S, HD), jnp.float32),
    )(q, kt, v)


def _k3_body(attn_ref, wo_ref, x_ref, w2_ref, wr_ref, h2_ref, logits_ref):
    a = attn_ref[...]
    proj = jnp.concatenate(
        [jnp.dot(a[:, g * (H * HD):(g + 1) * (H * HD)], wo_ref[g],
                 preferred_element_type=jnp.float32) for g in range(G)],
        axis=-1)
    x2 = x_ref[...] + proj
    var = jnp.mean(x2 * x2, axis=-1, keepdims=True)
    h2 = w2_ref[...][None, :] * (x2 * jax.lax.rsqrt(var + EPS))
    h2_ref[...] = h2
    logits_ref[...] = jnp.dot(h2, wr_ref[...], preferred_element_type=jnp.float32)


def _k3(attn, wo, x2d, norm2_w, wr):
    return pl.pallas_call(
        _k3_body,
        grid=(NT,),
        in_specs=[
            pl.BlockSpec((TT, D), lambda t: (t, 0)),
            pl.BlockSpec((G, H * HD, H * HD), lambda t: (0, 0, 0)),
            pl.BlockSpec((TT, D), lambda t: (t, 0)),
            pl.BlockSpec((D,), lambda t: (0,)),
            pl.BlockSpec((D, E), lambda t: (0, 0)),
        ],
        out_specs=[
            pl.BlockSpec((TT, D), lambda t: (t, 0)),
            pl.BlockSpec((TT, E), lambda t: (t, 0)),
        ],
        out_shape=[
            jax.ShapeDtypeStruct((S, D), jnp.float32),
            jax.ShapeDtypeStruct((S, E), jnp.float32),
        ],
    )(attn, wo, x2d, norm2_w, wr)


CH = 512


def _k4_body(logits_ref, d1_ref, d2_ref, g1_ref, g2_ref, te_ref, lb_ref):
    logits = logits_ref[...]
    m = jnp.max(logits, axis=-1, keepdims=True)
    ex = jnp.exp(logits - m)
    probs = ex / jnp.sum(ex, axis=-1, keepdims=True)
    lanes = jax.lax.broadcasted_iota(jnp.int32, (S, E), 1)
    m1 = jnp.max(probs, axis=-1, keepdims=True)
    i1 = jnp.min(jnp.where(probs == m1, lanes, E), axis=-1, keepdims=True)
    sel1 = lanes == i1
    p2 = jnp.where(sel1, -1.0, probs)
    m2 = jnp.max(p2, axis=-1, keepdims=True)
    i2 = jnp.min(jnp.where(p2 == m2, lanes, E), axis=-1, keepdims=True)
    sel2 = lanes == i2
    denom = m1 + m2
    g1_ref[...] = m1 / denom
    g2_ref[...] = m2 / denom

    oh1 = sel1.astype(jnp.float32)
    oh2 = sel2.astype(jnp.float32)
    f = jnp.sum(oh1 + oh2, axis=0) / S
    P = jnp.sum(probs, axis=0) / S
    lb_ref[0, 0] = (E / K) * jnp.sum(f * P)

    r = jax.lax.broadcasted_iota(jnp.int32, (CH, CH), 0)
    c = jax.lax.broadcasted_iota(jnp.int32, (CH, CH), 1)
    tri = jnp.where(r > c, 1.0, 0.0)
    carry = jnp.zeros((1, E), jnp.float32)
    ranks = []
    for oh in (oh1, oh2):
        parts = []
        for ch in range(S // CH):
            blk = oh[ch * CH:(ch + 1) * CH, :]
            parts.append(jnp.dot(tri, blk, preferred_element_type=jnp.float32)
                         + carry)
            carry = carry + jnp.sum(blk, axis=0, keepdims=True)
        ranks.append(jnp.concatenate(parts, axis=0))
    counts = carry

    blocks = jnp.floor((counts + (BM - 1)) / BM)
    ru = jax.lax.broadcasted_iota(jnp.int32, (E, E), 0)
    cu = jax.lax.broadcasted_iota(jnp.int32, (E, E), 1)
    triu = jnp.where(ru < cu, 1.0, 0.0)
    off = BM * jnp.dot(blocks, triu, preferred_element_type=jnp.float32)

    d1 = jnp.sum(oh1 * (off + ranks[0]), axis=-1, keepdims=True)
    d2 = jnp.sum(oh2 * (off + ranks[1]), axis=-1, keepdims=True)
    d1_ref[...] = d1.astype(jnp.int32)
    d2_ref[...] = d2.astype(jnp.int32)

    ti = jax.lax.broadcasted_iota(jnp.int32, (8, NTILES), 1).astype(jnp.float32)
    offc = jnp.broadcast_to(off.reshape(E, 1) / BM, (E, NTILES))
    blkc = jnp.broadcast_to(blocks.reshape(E, 1), (E, NTILES))
    ind = jnp.where(jnp.logical_and(ti >= offc, ti < offc + blkc), 1.0, 0.0)
    eid = jnp.broadcast_to(
        jax.lax.broadcasted_iota(jnp.int32, (E, 1), 0).astype(jnp.float32),
        (E, NTILES))
    any_ind = jnp.sum(ind, axis=0, keepdims=True)
    te = jnp.sum(ind * eid, axis=0, keepdims=True) + 7.0 * (1.0 - any_ind)
    te_ref[...] = jnp.concatenate([te, any_ind], axis=0).astype(jnp.int32)


def _k4(logits):
    return pl.pallas_call(
        _k4_body,
        in_specs=[pl.BlockSpec((S, E), lambda: (0, 0))],
        out_specs=[
            pl.BlockSpec((S, 1), lambda: (0, 0)),
            pl.BlockSpec((S, 1), lambda: (0, 0)),
            pl.BlockSpec((S, 1), lambda: (0, 0)),
            pl.BlockSpec((S, 1), lambda: (0, 0)),
            pl.BlockSpec((2, NTILES), lambda: (0, 0)),
            pl.BlockSpec(memory_space=pltpu.SMEM),
        ],
        out_shape=[
            jax.ShapeDtypeStruct((S, 1), jnp.int32),
            jax.ShapeDtypeStruct((S, 1), jnp.int32),
            jax.ShapeDtypeStruct((S, 1), jnp.float32),
            jax.ShapeDtypeStruct((S, 1), jnp.float32),
            jax.ShapeDtypeStruct((2, NTILES), jnp.int32),
            jax.ShapeDtypeStruct((1, 1), jnp.float32),
        ],
    )(logits)


def _gelu(x):
    c = np.sqrt(2.0 / np.pi).astype(np.float32)
    return 0.5 * x * (1.0 + jnp.tanh(c * (x + 0.044715 * x * x * x)))


def _k5_body(te_ref, xs_ref, w1_ref, w2_ref, ys_ref):
    i = pl.program_id(0)

    @pl.when(te_ref[1, i] != 0)
    def _():
        x = xs_ref[...].astype(jnp.bfloat16)
        h = _gelu(jnp.dot(x, w1_ref[0], preferred_element_type=jnp.float32))
        ys_ref[...] = jnp.dot(h.astype(jnp.bfloat16), w2_ref[0],
                              preferred_element_type=jnp.float32)


def _k5(xs, w1, w2, te):
    grid_spec = pltpu.PrefetchScalarGridSpec(
        num_scalar_prefetch=1,
        grid=(NTILES,),
        in_specs=[
            pl.BlockSpec((BM, D), lambda i, te: (i, 0)),
            pl.BlockSpec((1, D, DFF), lambda i, te: (te[0, i], 0, 0)),
            pl.BlockSpec((1, DFF, D), lambda i, te: (te[0, i], 0, 0)),
        ],
        out_specs=pl.BlockSpec((BM, D), lambda i, te: (i, 0)),
    )
    return pl.pallas_call(
        _k5_body,
        grid_spec=grid_spec,
        out_shape=jax.ShapeDtypeStruct((GPAD, D), jnp.float32),
    )(te, xs, w1, w2)


def _sc_wid():
    return lax.axis_index("s") * 2 + lax.axis_index("c")


def _scatter_body(h2_hbm, d1_hbm, d2_hbm, xs_hbm, rows_v, idx1_v, idx2_v, sem):
    base = _sc_wid() * CPW
    pltpu.sync_copy(h2_hbm.at[pl.ds(base, CPW)], rows_v)
    pltpu.sync_copy(d1_hbm.at[pl.ds(base, CPW)], idx1_v)
    pltpu.sync_copy(d2_hbm.at[pl.ds(base, CPW)], idx2_v)
    pltpu.async_copy(rows_v, xs_hbm.at[idx1_v], sem).wait()
    pltpu.async_copy(rows_v, xs_hbm.at[idx2_v], sem).wait()


def _sc_scatter(h2, d1, d2):
    mesh = plsc.VectorSubcoreMesh(core_axis_name="c", subcore_axis_name="s")
    kfn = functools.partial(
        pl.kernel, mesh=mesh,
        out_type=jax.ShapeDtypeStruct((GPAD, D), jnp.float32),
        scratch_types=[
            pltpu.VMEM((CPW, D), jnp.float32),
            pltpu.VMEM((CPW,), jnp.int32),
            pltpu.VMEM((CPW,), jnp.int32),
            pltpu.SemaphoreType.DMA,
        ],
    )(_scatter_body)
    return kfn(h2, d1, d2)


def _combine_body(ys_hbm, d1_hbm, d2_hbm, y1_hbm, y2_hbm,
                  buf1_v, buf2_v, idx1_v, idx2_v, sem1, sem2):
    base = _sc_wid() * CPW
    pltpu.sync_copy(d1_hbm.at[pl.ds(base, CPW)], idx1_v)
    pltpu.sync_copy(d2_hbm.at[pl.ds(base, CPW)], idx2_v)
    c1 = pltpu.async_copy(ys_hbm.at[idx1_v], buf1_v, sem1)
    c2 = pltpu.async_copy(ys_hbm.at[idx2_v], buf2_v, sem2)
    c1.wait()
    pltpu.sync_copy(buf1_v, y1_hbm.at[pl.ds(base, CPW)])
    c2.wait()
    pltpu.sync_copy(buf2_v, y2_hbm.at[pl.ds(base, CPW)])


def _sc_combine(ys, d1, d2):
    mesh = plsc.VectorSubcoreMesh(core_axis_name="c", subcore_axis_name="s")
    kfn = functools.partial(
        pl.kernel, mesh=mesh,
        out_type=[
            jax.ShapeDtypeStruct((S, D), jnp.float32),
            jax.ShapeDtypeStruct((S, D), jnp.float32),
        ],
        scratch_types=[
            pltpu.VMEM((CPW, D), jnp.float32),
            pltpu.VMEM((CPW, D), jnp.float32),
            pltpu.VMEM((CPW,), jnp.int32),
            pltpu.VMEM((CPW,), jnp.int32),
            pltpu.SemaphoreType.DMA,
            pltpu.SemaphoreType.DMA,
        ],
    )(_combine_body)
    return kfn(ys, d1, d2)


def _k6_body(y1_ref, y2_ref, g1_ref, g2_ref, out_ref):
    out_ref[...] = g1_ref[...] * y1_ref[...] + g2_ref[...] * y2_ref[...]


def _k6(y1, y2, g1, g2):
    return pl.pallas_call(
        _k6_body,
        grid=(NT,),
        in_specs=[
            pl.BlockSpec((TT, D), lambda t: (t, 0)),
            pl.BlockSpec((TT, D), lambda t: (t, 0)),
            pl.BlockSpec((TT, 1), lambda t: (t, 0)),
            pl.BlockSpec((TT, 1), lambda t: (t, 0)),
        ],
        out_specs=pl.BlockSpec((TT, D), lambda t: (t, 0)),
        out_shape=jax.ShapeDtypeStruct((S, D), jnp.float32),
    )(y1, y2, g1, g2)


def kernel(x, norm1_w, norm2_w, Wq, Wk, Wv, Wo, Wr, W1, W2):
    x2d = x.reshape(S, D)
    wq_all = Wq.transpose(1, 0, 2).reshape(D, D)
    wk_all = Wk.transpose(1, 0, 2).reshape(D, G * HD)
    wv_all = Wv.transpose(1, 0, 2).reshape(D, G * HD)
    wqkv = jnp.concatenate([wq_all, wk_all, wv_all], axis=1)

    half = HD // 2
    freqs = 1.0 / (10000.0 ** (jnp.arange(half, dtype=jnp.float32) / half))
    ang = jnp.arange(S, dtype=jnp.float32)[:, None] * freqs
    cos, sin = jnp.cos(ang), jnp.sin(ang)

    q, k, v = _k1(x2d, norm1_w, wqkv, cos, sin)
    q3 = q.reshape(S, NH, HD).transpose(1, 0, 2)
    kt = k.reshape(S, G, HD).transpose(1, 2, 0)
    v3 = v.reshape(S, G, HD).transpose(1, 0, 2)
    attn = _k2(q3, kt, v3)
    attn2d = attn.transpose(1, 0, 2).reshape(S, NH * HD)
    h2, logits = _k3(attn2d, Wo, x2d, norm2_w, Wr)
    d1, d2, g1, g2, te, lb = _k4(logits)
    d1f, d2f = d1.reshape(S), d2.reshape(S)
    xs = _sc_scatter(h2, d1f, d2f)
    ys = _k5(xs, W1.astype(jnp.bfloat16), W2.astype(jnp.bfloat16), te)
    y1, y2 = _sc_combine(ys, d1f, d2f)
    out = _k6(y1, y2, g1, g2)
    return out.reshape(B, S, D), lb[0, 0]


# speedup vs baseline: 2.0092x; 1.2324x over previous
import functools
from typing import Any

import jax
import jax.numpy as jnp
import numpy as np
from jax import lax
from jax.experimental import pallas as pl
from jax.experimental.pallas import tpu as pltpu
from jax.experimental.pallas import tpu_sc as plsc

B, S, D = 1, 2048, 768
G, H = 4, 3
HD = D // (G * H)
NH = G * H
E, K = 8, 2
DFF = 4 * D
EPS = 1e-6

TT = 256
NT = S // TT
FT = 1024
NF = DFF // FT

BM = 256
NTILES = (K * S) // BM + E
GPAD = NTILES * BM
CPW = S // 32


def _rope_cols(t, nheads, cos, sin):
    pieces = []
    for h in range(nheads):
        a = t[:, h * HD:h * HD + HD // 2]
        b = t[:, h * HD + HD // 2:(h + 1) * HD]
        pieces.append(a * cos - b * sin)
        pieces.append(a * sin + b * cos)
    return jnp.concatenate(pieces, axis=-1)


def _k1_body(x_ref, w_ref, wqkv_ref, cos_ref, sin_ref, q_ref, k_ref, v_ref):
    x = x_ref[...]
    var = jnp.mean(x * x, axis=-1, keepdims=True)
    h1 = w_ref[...][None, :] * (x * jax.lax.rsqrt(var + EPS))
    qkv = jnp.dot(h1, wqkv_ref[...], preferred_element_type=jnp.float32)
    cos, sin = cos_ref[...], sin_ref[...]
    q_ref[...] = (_rope_cols(qkv[:, :D], NH, cos, sin)
                  * (1.0 / np.sqrt(HD))).astype(jnp.bfloat16)
    k_ref[...] = _rope_cols(qkv[:, D:D + G * HD], G, cos, sin).astype(jnp.bfloat16)
    v_ref[...] = qkv[:, D + G * HD:].astype(jnp.bfloat16)


def _k1(x2d, norm1_w, wqkv, cos, sin):
    return pl.pallas_call(
        _k1_body,
        grid=(NT,),
        in_specs=[
            pl.BlockSpec((TT, D), lambda t: (t, 0)),
            pl.BlockSpec((D,), lambda t: (0,)),
            pl.BlockSpec((D, D + 2 * G * HD), lambda t: (0, 0)),
            pl.BlockSpec((TT, HD // 2), lambda t: (t, 0)),
            pl.BlockSpec((TT, HD // 2), lambda t: (t, 0)),
        ],
        out_specs=[
            pl.BlockSpec((TT, D), lambda t: (t, 0)),
            pl.BlockSpec((TT, G * HD), lambda t: (t, 0)),
            pl.BlockSpec((TT, G * HD), lambda t: (t, 0)),
        ],
        out_shape=[
            jax.ShapeDtypeStruct((S, D), jnp.bfloat16),
            jax.ShapeDtypeStruct((S, G * HD), jnp.bfloat16),
            jax.ShapeDtypeStruct((S, G * HD), jnp.bfloat16),
        ],
    )(x2d, norm1_w, wqkv, cos, sin)


HT = H * TT


def _k2_body(q_ref, kt_ref, v_ref, o_ref):
    t = pl.program_id(1)
    q = q_ref[0].reshape(HT, HD)

    def chunk(c, carry):
        acc, m, l = carry
        off = pl.multiple_of(c * TT, TT)
        kt = kt_ref[0, :, pl.ds(off, TT)]
        vc = v_ref[0, pl.ds(off, TT), :]
        sc = jnp.dot(q, kt, preferred_element_type=jnp.float32)
        rows = jax.lax.broadcasted_iota(jnp.int32, (HT, TT), 0)
        cols = jax.lax.broadcasted_iota(jnp.int32, (HT, TT), 1)
        sc = jnp.where(
            jnp.logical_or(c < t, cols <= (rows & (TT - 1))), sc, -1e9)
        m_new = jnp.maximum(m, jnp.max(sc, axis=-1, keepdims=True))
        p = jnp.exp(sc - m_new)
        alpha = jnp.exp(m - m_new)
        l = l * alpha + jnp.sum(p, axis=-1, keepdims=True)
        pv = jnp.dot(p.astype(jnp.bfloat16), vc,
                     preferred_element_type=jnp.float32)
        return acc * alpha + pv, m_new, l

    acc0 = jnp.zeros((HT, HD), jnp.float32)
    m0 = jnp.full((HT, 1), -1e30, jnp.float32)
    l0 = jnp.zeros((HT, 1), jnp.float32)
    acc, m, l = jax.lax.fori_loop(0, t + 1, chunk, (acc0, m0, l0))
    o_ref[0] = (acc / l).reshape(H, TT, HD)


def _k2(q, kt, v):
    return pl.pallas_call(
        _k2_body,
        grid=(G, NT),
        in_specs=[
            pl.BlockSpec((1, H, TT, HD), lambda g, t: (g, 0, t, 0)),
            pl.BlockSpec((1, HD, S), lambda g, t: (g, 0, 0)),
            pl.BlockSpec((1, S, HD), lambda g, t: (g, 0, 0)),
        ],
        out_specs=pl.BlockSpec((1, H, TT, HD), lambda g, t: (g, 0, t, 0)),
        out_shape=jax.ShapeDtypeStruct((G, H, S, HD), jnp.float32),
    )(q, kt, v)


def _k3_body(attn_ref, wo_ref, x_ref, w2_ref, wr_ref, h2_ref, logits_ref):
    a = attn_ref[...]
    proj = jnp.concatenate(
        [jnp.dot(a[:, g * (H * HD):(g + 1) * (H * HD)], wo_ref[g],
                 preferred_element_type=jnp.float32) for g in range(G)],
        axis=-1)
    x2 = x_ref[...] + proj
    var = jnp.mean(x2 * x2, axis=-1, keepdims=True)
    h2 = w2_ref[...][None, :] * (x2 * jax.lax.rsqrt(var + EPS))
    h2_ref[...] = h2
    logits_ref[...] = jnp.dot(h2, wr_ref[...], preferred_element_type=jnp.float32)


def _k3(attn, wo, x2d, norm2_w, wr):
    return pl.pallas_call(
        _k3_body,
        grid=(NT,),
        in_specs=[
            pl.BlockSpec((TT, D), lambda t: (t, 0)),
            pl.BlockSpec((G, H * HD, H * HD), lambda t: (0, 0, 0)),
            pl.BlockSpec((TT, D), lambda t: (t, 0)),
            pl.BlockSpec((D,), lambda t: (0,)),
            pl.BlockSpec((D, E), lambda t: (0, 0)),
        ],
        out_specs=[
            pl.BlockSpec((TT, D), lambda t: (t, 0)),
            pl.BlockSpec((TT, E), lambda t: (t, 0)),
        ],
        out_shape=[
            jax.ShapeDtypeStruct((S, D), jnp.float32),
            jax.ShapeDtypeStruct((S, E), jnp.float32),
        ],
    )(attn, wo, x2d, norm2_w, wr)


CH = 512


def _k4_body(logits_ref, d1_ref, d2_ref, g1_ref, g2_ref, te_ref, lb_ref):
    logits = logits_ref[...]
    m = jnp.max(logits, axis=-1, keepdims=True)
    ex = jnp.exp(logits - m)
    probs = ex / jnp.sum(ex, axis=-1, keepdims=True)
    lanes = jax.lax.broadcasted_iota(jnp.int32, (S, E), 1)
    m1 = jnp.max(probs, axis=-1, keepdims=True)
    i1 = jnp.min(jnp.where(probs == m1, lanes, E), axis=-1, keepdims=True)
    sel1 = lanes == i1
    p2 = jnp.where(sel1, -1.0, probs)
    m2 = jnp.max(p2, axis=-1, keepdims=True)
    i2 = jnp.min(jnp.where(p2 == m2, lanes, E), axis=-1, keepdims=True)
    sel2 = lanes == i2
    denom = m1 + m2
    g1_ref[...] = m1 / denom
    g2_ref[...] = m2 / denom

    oh1 = sel1.astype(jnp.float32)
    oh2 = sel2.astype(jnp.float32)
    f = jnp.sum(oh1 + oh2, axis=0) / S
    P = jnp.sum(probs, axis=0) / S
    lb_ref[0, 0] = (E / K) * jnp.sum(f * P)

    r = jax.lax.broadcasted_iota(jnp.int32, (CH, CH), 0)
    c = jax.lax.broadcasted_iota(jnp.int32, (CH, CH), 1)
    tri = jnp.where(r > c, 1.0, 0.0)
    carry = jnp.zeros((1, E), jnp.float32)
    ranks = []
    for oh in (oh1, oh2):
        parts = []
        for ch in range(S // CH):
            blk = oh[ch * CH:(ch + 1) * CH, :]
            parts.append(jnp.dot(tri, blk, preferred_element_type=jnp.float32)
                         + carry)
            carry = carry + jnp.sum(blk, axis=0, keepdims=True)
        ranks.append(jnp.concatenate(parts, axis=0))
    counts = carry

    blocks = jnp.floor((counts + (BM - 1)) / BM)
    ru = jax.lax.broadcasted_iota(jnp.int32, (E, E), 0)
    cu = jax.lax.broadcasted_iota(jnp.int32, (E, E), 1)
    triu = jnp.where(ru < cu, 1.0, 0.0)
    off = BM * jnp.dot(blocks, triu, preferred_element_type=jnp.float32)

    d1 = jnp.sum(oh1 * (off + ranks[0]), axis=-1, keepdims=True)
    d2 = jnp.sum(oh2 * (off + ranks[1]), axis=-1, keepdims=True)
    d1_ref[...] = d1.astype(jnp.int32)
    d2_ref[...] = d2.astype(jnp.int32)

    ti = jax.lax.broadcasted_iota(jnp.int32, (8, NTILES), 1).astype(jnp.float32)
    offc = jnp.broadcast_to(off.reshape(E, 1) / BM, (E, NTILES))
    blkc = jnp.broadcast_to(blocks.reshape(E, 1), (E, NTILES))
    ind = jnp.where(jnp.logical_and(ti >= offc, ti < offc + blkc), 1.0, 0.0)
    eid = jnp.broadcast_to(
        jax.lax.broadcasted_iota(jnp.int32, (E, 1), 0).astype(jnp.float32),
        (E, NTILES))
    any_ind = jnp.sum(ind, axis=0, keepdims=True)
    te = jnp.sum(ind * eid, axis=0, keepdims=True) + 7.0 * (1.0 - any_ind)
    te_ref[...] = jnp.concatenate([te, any_ind], axis=0).astype(jnp.int32)


def _k4(logits):
    return pl.pallas_call(
        _k4_body,
        in_specs=[pl.BlockSpec((S, E), lambda: (0, 0))],
        out_specs=[
            pl.BlockSpec((S, 1), lambda: (0, 0)),
            pl.BlockSpec((S, 1), lambda: (0, 0)),
            pl.BlockSpec((S, 1), lambda: (0, 0)),
            pl.BlockSpec((S, 1), lambda: (0, 0)),
            pl.BlockSpec((2, NTILES), lambda: (0, 0)),
            pl.BlockSpec(memory_space=pltpu.SMEM),
        ],
        out_shape=[
            jax.ShapeDtypeStruct((S, 1), jnp.int32),
            jax.ShapeDtypeStruct((S, 1), jnp.int32),
            jax.ShapeDtypeStruct((S, 1), jnp.float32),
            jax.ShapeDtypeStruct((S, 1), jnp.float32),
            jax.ShapeDtypeStruct((2, NTILES), jnp.int32),
            jax.ShapeDtypeStruct((1, 1), jnp.float32),
        ],
    )(logits)


def _gelu(x):
    c = np.sqrt(2.0 / np.pi).astype(np.float32)
    return 0.5 * x * (1.0 + jnp.tanh(c * (x + 0.044715 * x * x * x)))


def _k5_body(te_ref, xs_ref, w1_ref, w2_ref, ys_ref):
    i = pl.program_id(0)

    @pl.when(te_ref[1, i] != 0)
    def _():
        x = xs_ref[...].astype(jnp.bfloat16)
        h = _gelu(jnp.dot(x, w1_ref[0], preferred_element_type=jnp.float32))
        ys_ref[...] = jnp.dot(h.astype(jnp.bfloat16), w2_ref[0],
                              preferred_element_type=jnp.float32)


def _k5(xs, w1, w2, te):
    grid_spec = pltpu.PrefetchScalarGridSpec(
        num_scalar_prefetch=1,
        grid=(NTILES,),
        in_specs=[
            pl.BlockSpec((BM, D), lambda i, te: (i, 0)),
            pl.BlockSpec((1, D, DFF), lambda i, te: (te[0, i], 0, 0)),
            pl.BlockSpec((1, DFF, D), lambda i, te: (te[0, i], 0, 0)),
        ],
        out_specs=pl.BlockSpec((BM, D), lambda i, te: (i, 0)),
    )
    return pl.pallas_call(
        _k5_body,
        grid_spec=grid_spec,
        out_shape=jax.ShapeDtypeStruct((GPAD, D), jnp.float32),
    )(te, xs, w1, w2)


def _sc_wid():
    return lax.axis_index("s") * 2 + lax.axis_index("c")


def _scatter_body(h2_hbm, d1_hbm, d2_hbm, xs_hbm, rows_v, idx1_v, idx2_v, sem):
    base = _sc_wid() * CPW
    pltpu.sync_copy(h2_hbm.at[pl.ds(base, CPW)], rows_v)
    pltpu.sync_copy(d1_hbm.at[pl.ds(base, CPW)], idx1_v)
    pltpu.sync_copy(d2_hbm.at[pl.ds(base, CPW)], idx2_v)
    pltpu.async_copy(rows_v, xs_hbm.at[idx1_v], sem).wait()
    pltpu.async_copy(rows_v, xs_hbm.at[idx2_v], sem).wait()


def _sc_scatter(h2, d1, d2):
    mesh = plsc.VectorSubcoreMesh(core_axis_name="c", subcore_axis_name="s")
    kfn = functools.partial(
        pl.kernel, mesh=mesh,
        out_type=jax.ShapeDtypeStruct((GPAD, D), jnp.float32),
        scratch_types=[
            pltpu.VMEM((CPW, D), jnp.float32),
            pltpu.VMEM((CPW,), jnp.int32),
            pltpu.VMEM((CPW,), jnp.int32),
            pltpu.SemaphoreType.DMA,
        ],
    )(_scatter_body)
    return kfn(h2, d1, d2)


def _combine_body(ys_hbm, d1_hbm, d2_hbm, y1_hbm, y2_hbm,
                  buf1_v, buf2_v, idx1_v, idx2_v, sem1, sem2):
    base = _sc_wid() * CPW
    pltpu.sync_copy(d1_hbm.at[pl.ds(base, CPW)], idx1_v)
    pltpu.sync_copy(d2_hbm.at[pl.ds(base, CPW)], idx2_v)
    c1 = pltpu.async_copy(ys_hbm.at[idx1_v], buf1_v, sem1)
    c2 = pltpu.async_copy(ys_hbm.at[idx2_v], buf2_v, sem2)
    c1.wait()
    pltpu.sync_copy(buf1_v, y1_hbm.at[pl.ds(base, CPW)])
    c2.wait()
    pltpu.sync_copy(buf2_v, y2_hbm.at[pl.ds(base, CPW)])


def _sc_combine(ys, d1, d2):
    mesh = plsc.VectorSubcoreMesh(core_axis_name="c", subcore_axis_name="s")
    kfn = functools.partial(
        pl.kernel, mesh=mesh,
        out_type=[
            jax.ShapeDtypeStruct((S, D), jnp.float32),
            jax.ShapeDtypeStruct((S, D), jnp.float32),
        ],
        scratch_types=[
            pltpu.VMEM((CPW, D), jnp.float32),
            pltpu.VMEM((CPW, D), jnp.float32),
            pltpu.VMEM((CPW,), jnp.int32),
            pltpu.VMEM((CPW,), jnp.int32),
            pltpu.SemaphoreType.DMA,
            pltpu.SemaphoreType.DMA,
        ],
    )(_combine_body)
    return kfn(ys, d1, d2)


def _k6_body(y1_ref, y2_ref, g1_ref, g2_ref, out_ref):
    out_ref[...] = g1_ref[...] * y1_ref[...] + g2_ref[...] * y2_ref[...]


def _k6(y1, y2, g1, g2):
    return pl.pallas_call(
        _k6_body,
        grid=(NT,),
        in_specs=[
            pl.BlockSpec((TT, D), lambda t: (t, 0)),
            pl.BlockSpec((TT, D), lambda t: (t, 0)),
            pl.BlockSpec((TT, 1), lambda t: (t, 0)),
            pl.BlockSpec((TT, 1), lambda t: (t, 0)),
        ],
        out_specs=pl.BlockSpec((TT, D), lambda t: (t, 0)),
        out_shape=jax.ShapeDtypeStruct((S, D), jnp.float32),
    )(y1, y2, g1, g2)


def kernel(x, norm1_w, norm2_w, Wq, Wk, Wv, Wo, Wr, W1, W2):
    x2d = x.reshape(S, D)
    wq_all = Wq.transpose(1, 0, 2).reshape(D, D)
    wk_all = Wk.transpose(1, 0, 2).reshape(D, G * HD)
    wv_all = Wv.transpose(1, 0, 2).reshape(D, G * HD)
    wqkv = jnp.concatenate([wq_all, wk_all, wv_all], axis=1)

    half = HD // 2
    freqs = 1.0 / (10000.0 ** (jnp.arange(half, dtype=jnp.float32) / half))
    ang = jnp.arange(S, dtype=jnp.float32)[:, None] * freqs
    cos, sin = jnp.cos(ang), jnp.sin(ang)

    q, k, v = _k1(x2d, norm1_w, wqkv, cos, sin)
    q3 = q.reshape(S, G, H, HD).transpose(1, 2, 0, 3)
    kt = k.reshape(S, G, HD).transpose(1, 2, 0)
    v3 = v.reshape(S, G, HD).transpose(1, 0, 2)
    attn = _k2(q3, kt, v3)
    attn2d = attn.transpose(2, 0, 1, 3).reshape(S, NH * HD)
    h2, logits = _k3(attn2d, Wo, x2d, norm2_w, Wr)
    d1, d2, g1, g2, te, lb = _k4(logits)
    d1f, d2f = d1.reshape(S), d2.reshape(S)
    xs = _sc_scatter(h2, d1f, d2f)
    ys = _k5(xs, W1.astype(jnp.bfloat16), W2.astype(jnp.bfloat16), te)
    y1, y2 = _sc_combine(ys, d1f, d2f)
    out = _k6(y1, y2, g1, g2)
    return out.reshape(B, S, D), lb[0, 0]


# speedup vs baseline: 2.0252x; 1.0079x over previous
import functools
from typing import Any

import jax
import jax.numpy as jnp
import numpy as np
from jax import lax
from jax.experimental import pallas as pl
from jax.experimental.pallas import tpu as pltpu
from jax.experimental.pallas import tpu_sc as plsc

B, S, D = 1, 2048, 768
G, H = 4, 3
HD = D // (G * H)
NH = G * H
E, K = 8, 2
DFF = 4 * D
EPS = 1e-6

TT = 256
NT = S // TT
FT = 1024
NF = DFF // FT

BM = 256
NTILES = (K * S) // BM + E
GPAD = NTILES * BM
CPW = S // 32


def _rope_cols(t, nheads, cos, sin):
    pieces = []
    for h in range(nheads):
        a = t[:, h * HD:h * HD + HD // 2]
        b = t[:, h * HD + HD // 2:(h + 1) * HD]
        pieces.append(a * cos - b * sin)
        pieces.append(a * sin + b * cos)
    return jnp.concatenate(pieces, axis=-1)


def _k1_body(x_ref, w_ref, wqkv_ref, cos_ref, sin_ref, q_ref, k_ref, v_ref):
    x = x_ref[...]
    var = jnp.mean(x * x, axis=-1, keepdims=True)
    h1 = w_ref[...][None, :] * (x * jax.lax.rsqrt(var + EPS))
    qkv = jnp.dot(h1, wqkv_ref[...], preferred_element_type=jnp.float32)
    cos, sin = cos_ref[...], sin_ref[...]
    q_ref[...] = (_rope_cols(qkv[:, :D], NH, cos, sin)
                  * (1.0 / np.sqrt(HD))).astype(jnp.bfloat16)
    k_ref[...] = _rope_cols(qkv[:, D:D + G * HD], G, cos, sin).astype(jnp.bfloat16)
    v_ref[...] = qkv[:, D + G * HD:].astype(jnp.bfloat16)


def _k1(x2d, norm1_w, wqkv, cos, sin):
    return pl.pallas_call(
        _k1_body,
        grid=(NT,),
        in_specs=[
            pl.BlockSpec((TT, D), lambda t: (t, 0)),
            pl.BlockSpec((D,), lambda t: (0,)),
            pl.BlockSpec((D, D + 2 * G * HD), lambda t: (0, 0)),
            pl.BlockSpec((TT, HD // 2), lambda t: (t, 0)),
            pl.BlockSpec((TT, HD // 2), lambda t: (t, 0)),
        ],
        out_specs=[
            pl.BlockSpec((TT, D), lambda t: (t, 0)),
            pl.BlockSpec((TT, G * HD), lambda t: (t, 0)),
            pl.BlockSpec((TT, G * HD), lambda t: (t, 0)),
        ],
        out_shape=[
            jax.ShapeDtypeStruct((S, D), jnp.bfloat16),
            jax.ShapeDtypeStruct((S, G * HD), jnp.bfloat16),
            jax.ShapeDtypeStruct((S, G * HD), jnp.bfloat16),
        ],
    )(x2d, norm1_w, wqkv, cos, sin)


HT = H * TT


def _k2_body(q_ref, kt_ref, v_ref, o_ref):
    t = pl.program_id(1)
    q = q_ref[0].reshape(HT, HD)

    def chunk(c, carry, masked):
        acc, m, l = carry
        off = pl.multiple_of(c * TT, TT)
        kt = kt_ref[0, :, pl.ds(off, TT)]
        vc = v_ref[0, pl.ds(off, TT), :]
        sc = jnp.dot(q, kt, preferred_element_type=jnp.float32)
        if masked:
            rows = jax.lax.broadcasted_iota(jnp.int32, (HT, TT), 0)
            cols = jax.lax.broadcasted_iota(jnp.int32, (HT, TT), 1)
            sc = jnp.where(cols <= (rows & (TT - 1)), sc, -1e9)
        m_new = jnp.maximum(m, jnp.max(sc, axis=-1, keepdims=True))
        p = jnp.exp(sc - m_new)
        alpha = jnp.exp(m - m_new)
        l = l * alpha + jnp.sum(p, axis=-1, keepdims=True)
        pv = jnp.dot(p.astype(jnp.bfloat16), vc,
                     preferred_element_type=jnp.float32)
        return acc * alpha + pv, m_new, l

    acc0 = jnp.zeros((HT, HD), jnp.float32)
    m0 = jnp.full((HT, 1), -1e30, jnp.float32)
    l0 = jnp.zeros((HT, 1), jnp.float32)
    carry = jax.lax.fori_loop(
        0, t, lambda c, cr: chunk(c, cr, False), (acc0, m0, l0))
    acc, m, l = chunk(t, carry, True)
    o_ref[0] = (acc / l).reshape(H, TT, HD)


def _k2(q, kt, v):
    return pl.pallas_call(
        _k2_body,
        grid=(G, NT),
        in_specs=[
            pl.BlockSpec((1, H, TT, HD), lambda g, t: (g, 0, t, 0)),
            pl.BlockSpec((1, HD, S), lambda g, t: (g, 0, 0)),
            pl.BlockSpec((1, S, HD), lambda g, t: (g, 0, 0)),
        ],
        out_specs=pl.BlockSpec((1, H, TT, HD), lambda g, t: (g, 0, t, 0)),
        out_shape=jax.ShapeDtypeStruct((G, H, S, HD), jnp.float32),
    )(q, kt, v)


def _k3_body(attn_ref, wo_ref, x_ref, w2_ref, wr_ref, h2_ref, logits_ref):
    a = attn_ref[...]
    proj = jnp.concatenate(
        [jnp.dot(a[:, g * (H * HD):(g + 1) * (H * HD)], wo_ref[g],
                 preferred_element_type=jnp.float32) for g in range(G)],
        axis=-1)
    x2 = x_ref[...] + proj
    var = jnp.mean(x2 * x2, axis=-1, keepdims=True)
    h2 = w2_ref[...][None, :] * (x2 * jax.lax.rsqrt(var + EPS))
    h2_ref[...] = h2
    logits_ref[...] = jnp.dot(h2, wr_ref[...], preferred_element_type=jnp.float32)


def _k3(attn, wo, x2d, norm2_w, wr):
    return pl.pallas_call(
        _k3_body,
        grid=(NT,),
        in_specs=[
            pl.BlockSpec((TT, D), lambda t: (t, 0)),
            pl.BlockSpec((G, H * HD, H * HD), lambda t: (0, 0, 0)),
            pl.BlockSpec((TT, D), lambda t: (t, 0)),
            pl.BlockSpec((D,), lambda t: (0,)),
            pl.BlockSpec((D, E), lambda t: (0, 0)),
        ],
        out_specs=[
            pl.BlockSpec((TT, D), lambda t: (t, 0)),
            pl.BlockSpec((TT, E), lambda t: (t, 0)),
        ],
        out_shape=[
            jax.ShapeDtypeStruct((S, D), jnp.float32),
            jax.ShapeDtypeStruct((S, E), jnp.float32),
        ],
    )(attn, wo, x2d, norm2_w, wr)


CH = 512


def _k4_body(logits_ref, d1_ref, d2_ref, g1_ref, g2_ref, te_ref, lb_ref):
    logits = logits_ref[...]
    m = jnp.max(logits, axis=-1, keepdims=True)
    ex = jnp.exp(logits - m)
    probs = ex / jnp.sum(ex, axis=-1, keepdims=True)
    lanes = jax.lax.broadcasted_iota(jnp.int32, (S, E), 1)
    m1 = jnp.max(probs, axis=-1, keepdims=True)
    i1 = jnp.min(jnp.where(probs == m1, lanes, E), axis=-1, keepdims=True)
    sel1 = lanes == i1
    p2 = jnp.where(sel1, -1.0, probs)
    m2 = jnp.max(p2, axis=-1, keepdims=True)
    i2 = jnp.min(jnp.where(p2 == m2, lanes, E), axis=-1, keepdims=True)
    sel2 = lanes == i2
    denom = m1 + m2
    g1_ref[...] = m1 / denom
    g2_ref[...] = m2 / denom

    oh1 = sel1.astype(jnp.float32)
    oh2 = sel2.astype(jnp.float32)
    f = jnp.sum(oh1 + oh2, axis=0) / S
    P = jnp.sum(probs, axis=0) / S
    lb_ref[0, 0] = (E / K) * jnp.sum(f * P)

    r = jax.lax.broadcasted_iota(jnp.int32, (CH, CH), 0)
    c = jax.lax.broadcasted_iota(jnp.int32, (CH, CH), 1)
    tri = jnp.where(r > c, 1.0, 0.0)
    carry = jnp.zeros((1, E), jnp.float32)
    ranks = []
    for oh in (oh1, oh2):
        parts = []
        for ch in range(S // CH):
            blk = oh[ch * CH:(ch + 1) * CH, :]
            parts.append(jnp.dot(tri, blk, preferred_element_type=jnp.float32)
                         + carry)
            carry = carry + jnp.sum(blk, axis=0, keepdims=True)
        ranks.append(jnp.concatenate(parts, axis=0))
    counts = carry

    blocks = jnp.floor((counts + (BM - 1)) / BM)
    ru = jax.lax.broadcasted_iota(jnp.int32, (E, E), 0)
    cu = jax.lax.broadcasted_iota(jnp.int32, (E, E), 1)
    triu = jnp.where(ru < cu, 1.0, 0.0)
    off = BM * jnp.dot(blocks, triu, preferred_element_type=jnp.float32)

    d1 = jnp.sum(oh1 * (off + ranks[0]), axis=-1, keepdims=True)
    d2 = jnp.sum(oh2 * (off + ranks[1]), axis=-1, keepdims=True)
    d1_ref[...] = d1.astype(jnp.int32)
    d2_ref[...] = d2.astype(jnp.int32)

    ti = jax.lax.broadcasted_iota(jnp.int32, (8, NTILES), 1).astype(jnp.float32)
    offc = jnp.broadcast_to(off.reshape(E, 1) / BM, (E, NTILES))
    blkc = jnp.broadcast_to(blocks.reshape(E, 1), (E, NTILES))
    ind = jnp.where(jnp.logical_and(ti >= offc, ti < offc + blkc), 1.0, 0.0)
    eid = jnp.broadcast_to(
        jax.lax.broadcasted_iota(jnp.int32, (E, 1), 0).astype(jnp.float32),
        (E, NTILES))
    any_ind = jnp.sum(ind, axis=0, keepdims=True)
    te = jnp.sum(ind * eid, axis=0, keepdims=True) + 7.0 * (1.0 - any_ind)
    te_ref[...] = jnp.concatenate([te, any_ind], axis=0).astype(jnp.int32)


def _k4(logits):
    return pl.pallas_call(
        _k4_body,
        in_specs=[pl.BlockSpec((S, E), lambda: (0, 0))],
        out_specs=[
            pl.BlockSpec((S, 1), lambda: (0, 0)),
            pl.BlockSpec((S, 1), lambda: (0, 0)),
            pl.BlockSpec((S, 1), lambda: (0, 0)),
            pl.BlockSpec((S, 1), lambda: (0, 0)),
            pl.BlockSpec((2, NTILES), lambda: (0, 0)),
            pl.BlockSpec(memory_space=pltpu.SMEM),
        ],
        out_shape=[
            jax.ShapeDtypeStruct((S, 1), jnp.int32),
            jax.ShapeDtypeStruct((S, 1), jnp.int32),
            jax.ShapeDtypeStruct((S, 1), jnp.float32),
            jax.ShapeDtypeStruct((S, 1), jnp.float32),
            jax.ShapeDtypeStruct((2, NTILES), jnp.int32),
            jax.ShapeDtypeStruct((1, 1), jnp.float32),
        ],
    )(logits)


def _gelu(x):
    c = np.sqrt(2.0 / np.pi).astype(np.float32)
    return 0.5 * x * (1.0 + jnp.tanh(c * (x + 0.044715 * x * x * x)))


def _k5_body(te_ref, xs_ref, w1_ref, w2_ref, ys_ref):
    i = pl.program_id(0)

    @pl.when(te_ref[1, i] != 0)
    def _():
        x = xs_ref[...].astype(jnp.bfloat16)
        h = jnp.dot(x, w1_ref[0], preferred_element_type=jnp.float32)
        g = _gelu(h.astype(jnp.bfloat16)).astype(jnp.bfloat16)
        ys_ref[...] = jnp.dot(g, w2_ref[0],
                              preferred_element_type=jnp.float32)


def _k5(xs, w1, w2, te):
    grid_spec = pltpu.PrefetchScalarGridSpec(
        num_scalar_prefetch=1,
        grid=(NTILES,),
        in_specs=[
            pl.BlockSpec((BM, D), lambda i, te: (i, 0)),
            pl.BlockSpec((1, D, DFF), lambda i, te: (te[0, i], 0, 0)),
            pl.BlockSpec((1, DFF, D), lambda i, te: (te[0, i], 0, 0)),
        ],
        out_specs=pl.BlockSpec((BM, D), lambda i, te: (i, 0)),
    )
    return pl.pallas_call(
        _k5_body,
        grid_spec=grid_spec,
        out_shape=jax.ShapeDtypeStruct((GPAD, D), jnp.float32),
    )(te, xs, w1, w2)


def _sc_wid():
    return lax.axis_index("s") * 2 + lax.axis_index("c")


def _scatter_body(h2_hbm, d1_hbm, d2_hbm, xs_hbm, rows_v, idx1_v, idx2_v, sem):
    base = _sc_wid() * CPW
    pltpu.sync_copy(h2_hbm.at[pl.ds(base, CPW)], rows_v)
    pltpu.sync_copy(d1_hbm.at[pl.ds(base, CPW)], idx1_v)
    pltpu.sync_copy(d2_hbm.at[pl.ds(base, CPW)], idx2_v)
    pltpu.async_copy(rows_v, xs_hbm.at[idx1_v], sem).wait()
    pltpu.async_copy(rows_v, xs_hbm.at[idx2_v], sem).wait()


def _sc_scatter(h2, d1, d2):
    mesh = plsc.VectorSubcoreMesh(core_axis_name="c", subcore_axis_name="s")
    kfn = functools.partial(
        pl.kernel, mesh=mesh,
        out_type=jax.ShapeDtypeStruct((GPAD, D), jnp.float32),
        scratch_types=[
            pltpu.VMEM((CPW, D), jnp.float32),
            pltpu.VMEM((CPW,), jnp.int32),
            pltpu.VMEM((CPW,), jnp.int32),
            pltpu.SemaphoreType.DMA,
        ],
    )(_scatter_body)
    return kfn(h2, d1, d2)


def _combine_body(ys_hbm, d1_hbm, d2_hbm, y1_hbm, y2_hbm,
                  buf1_v, buf2_v, idx1_v, idx2_v, sem1, sem2):
    base = _sc_wid() * CPW
    pltpu.sync_copy(d1_hbm.at[pl.ds(base, CPW)], idx1_v)
    pltpu.sync_copy(d2_hbm.at[pl.ds(base, CPW)], idx2_v)
    c1 = pltpu.async_copy(ys_hbm.at[idx1_v], buf1_v, sem1)
    c2 = pltpu.async_copy(ys_hbm.at[idx2_v], buf2_v, sem2)
    c1.wait()
    pltpu.sync_copy(buf1_v, y1_hbm.at[pl.ds(base, CPW)])
    c2.wait()
    pltpu.sync_copy(buf2_v, y2_hbm.at[pl.ds(base, CPW)])


def _sc_combine(ys, d1, d2):
    mesh = plsc.VectorSubcoreMesh(core_axis_name="c", subcore_axis_name="s")
    kfn = functools.partial(
        pl.kernel, mesh=mesh,
        out_type=[
            jax.ShapeDtypeStruct((S, D), jnp.float32),
            jax.ShapeDtypeStruct((S, D), jnp.float32),
        ],
        scratch_types=[
            pltpu.VMEM((CPW, D), jnp.float32),
            pltpu.VMEM((CPW, D), jnp.float32),
            pltpu.VMEM((CPW,), jnp.int32),
            pltpu.VMEM((CPW,), jnp.int32),
            pltpu.SemaphoreType.DMA,
            pltpu.SemaphoreType.DMA,
        ],
    )(_combine_body)
    return kfn(ys, d1, d2)


def _k6_body(y1_ref, y2_ref, g1_ref, g2_ref, out_ref):
    out_ref[...] = g1_ref[...] * y1_ref[...] + g2_ref[...] * y2_ref[...]


def _k6(y1, y2, g1, g2):
    return pl.pallas_call(
        _k6_body,
        grid=(NT,),
        in_specs=[
            pl.BlockSpec((TT, D), lambda t: (t, 0)),
            pl.BlockSpec((TT, D), lambda t: (t, 0)),
            pl.BlockSpec((TT, 1), lambda t: (t, 0)),
            pl.BlockSpec((TT, 1), lambda t: (t, 0)),
        ],
        out_specs=pl.BlockSpec((TT, D), lambda t: (t, 0)),
        out_shape=jax.ShapeDtypeStruct((S, D), jnp.float32),
    )(y1, y2, g1, g2)


def kernel(x, norm1_w, norm2_w, Wq, Wk, Wv, Wo, Wr, W1, W2):
    x2d = x.reshape(S, D)
    wq_all = Wq.transpose(1, 0, 2).reshape(D, D)
    wk_all = Wk.transpose(1, 0, 2).reshape(D, G * HD)
    wv_all = Wv.transpose(1, 0, 2).reshape(D, G * HD)
    wqkv = jnp.concatenate([wq_all, wk_all, wv_all], axis=1)

    half = HD // 2
    freqs = 1.0 / (10000.0 ** (jnp.arange(half, dtype=jnp.float32) / half))
    ang = jnp.arange(S, dtype=jnp.float32)[:, None] * freqs
    cos, sin = jnp.cos(ang), jnp.sin(ang)

    q, k, v = _k1(x2d, norm1_w, wqkv, cos, sin)
    q3 = q.reshape(S, G, H, HD).transpose(1, 2, 0, 3)
    kt = k.reshape(S, G, HD).transpose(1, 2, 0)
    v3 = v.reshape(S, G, HD).transpose(1, 0, 2)
    attn = _k2(q3, kt, v3)
    attn2d = attn.transpose(2, 0, 1, 3).reshape(S, NH * HD)
    h2, logits = _k3(attn2d, Wo, x2d, norm2_w, Wr)
    d1, d2, g1, g2, te, lb = _k4(logits)
    d1f, d2f = d1.reshape(S), d2.reshape(S)
    xs = _sc_scatter(h2, d1f, d2f)
    ys = _k5(xs, W1.astype(jnp.bfloat16), W2.astype(jnp.bfloat16), te)
    y1, y2 = _sc_combine(ys, d1f, d2f)
    out = _k6(y1, y2, g1, g2)
    return out.reshape(B, S, D), lb[0, 0]


# speedup vs baseline: 2.2832x; 1.1274x over previous
import functools
from typing import Any

import jax
import jax.numpy as jnp
import numpy as np
from jax import lax
from jax.experimental import pallas as pl
from jax.experimental.pallas import tpu as pltpu
from jax.experimental.pallas import tpu_sc as plsc

B, S, D = 1, 2048, 768
G, H = 4, 3
HD = D // (G * H)
NH = G * H
E, K = 8, 2
DFF = 4 * D
EPS = 1e-6

TT = 256
NT = S // TT
FT = 1024
NF = DFF // FT

BM = 256
NTILES = (K * S) // BM + E
GPAD = NTILES * BM
CPW = S // 32


def _rope_cols(t, nheads, cos, sin):
    pieces = []
    for h in range(nheads):
        a = t[:, h * HD:h * HD + HD // 2]
        b = t[:, h * HD + HD // 2:(h + 1) * HD]
        pieces.append(a * cos - b * sin)
        pieces.append(a * sin + b * cos)
    return jnp.concatenate(pieces, axis=-1)


def _k1_body(x_ref, w_ref, wqkv_ref, cos_ref, sin_ref, q_ref, k_ref, v_ref):
    x = x_ref[...]
    var = jnp.mean(x * x, axis=-1, keepdims=True)
    h1 = w_ref[...][None, :] * (x * jax.lax.rsqrt(var + EPS))
    qkv = jnp.dot(h1, wqkv_ref[...], preferred_element_type=jnp.float32)
    cos, sin = cos_ref[...], sin_ref[...]
    q_ref[...] = (_rope_cols(qkv[:, :D], NH, cos, sin)
                  * (1.0 / np.sqrt(HD))).astype(jnp.bfloat16)
    k_ref[...] = _rope_cols(qkv[:, D:D + G * HD], G, cos, sin).astype(jnp.bfloat16)
    v_ref[...] = qkv[:, D + G * HD:].astype(jnp.bfloat16)


def _k1(x2d, norm1_w, wqkv, cos, sin):
    return pl.pallas_call(
        _k1_body,
        grid=(NT,),
        in_specs=[
            pl.BlockSpec((TT, D), lambda t: (t, 0)),
            pl.BlockSpec((D,), lambda t: (0,)),
            pl.BlockSpec((D, D + 2 * G * HD), lambda t: (0, 0)),
            pl.BlockSpec((TT, HD // 2), lambda t: (t, 0)),
            pl.BlockSpec((TT, HD // 2), lambda t: (t, 0)),
        ],
        out_specs=[
            pl.BlockSpec((TT, D), lambda t: (t, 0)),
            pl.BlockSpec((TT, G * HD), lambda t: (t, 0)),
            pl.BlockSpec((TT, G * HD), lambda t: (t, 0)),
        ],
        out_shape=[
            jax.ShapeDtypeStruct((S, D), jnp.bfloat16),
            jax.ShapeDtypeStruct((S, G * HD), jnp.bfloat16),
            jax.ShapeDtypeStruct((S, G * HD), jnp.bfloat16),
        ],
    )(x2d, norm1_w, wqkv, cos, sin)


HT = H * TT


def _k2_body(q_ref, kt_ref, v_ref, o_ref):
    t = pl.program_id(1)
    q = q_ref[0].reshape(HT, HD)

    def chunk(c, carry, masked):
        acc, m, l = carry
        off = pl.multiple_of(c * TT, TT)
        kt = kt_ref[0, :, pl.ds(off, TT)]
        vc = v_ref[0, pl.ds(off, TT), :]
        sc = jnp.dot(q, kt, preferred_element_type=jnp.float32)
        if masked:
            rows = jax.lax.broadcasted_iota(jnp.int32, (HT, TT), 0)
            cols = jax.lax.broadcasted_iota(jnp.int32, (HT, TT), 1)
            sc = jnp.where(cols <= (rows & (TT - 1)), sc, -1e9)
        m_new = jnp.maximum(m, jnp.max(sc, axis=-1, keepdims=True))
        p = jnp.exp(sc - m_new)
        alpha = jnp.exp(m - m_new)
        l = l * alpha + jnp.sum(p, axis=-1, keepdims=True)
        pv = jnp.dot(p.astype(jnp.bfloat16), vc,
                     preferred_element_type=jnp.float32)
        return acc * alpha + pv, m_new, l

    acc0 = jnp.zeros((HT, HD), jnp.float32)
    m0 = jnp.full((HT, 1), -1e30, jnp.float32)
    l0 = jnp.zeros((HT, 1), jnp.float32)
    carry = jax.lax.fori_loop(
        0, t, lambda c, cr: chunk(c, cr, False), (acc0, m0, l0))
    acc, m, l = chunk(t, carry, True)
    o_ref[0] = (acc / l).reshape(H, TT, HD)


def _k2(q, kt, v):
    return pl.pallas_call(
        _k2_body,
        grid=(G, NT),
        in_specs=[
            pl.BlockSpec((1, H, TT, HD), lambda g, t: (g, 0, t, 0)),
            pl.BlockSpec((1, HD, S), lambda g, t: (g, 0, 0)),
            pl.BlockSpec((1, S, HD), lambda g, t: (g, 0, 0)),
        ],
        out_specs=pl.BlockSpec((1, H, TT, HD), lambda g, t: (g, 0, t, 0)),
        out_shape=jax.ShapeDtypeStruct((G, H, S, HD), jnp.float32),
    )(q, kt, v)


def _k3_body(attn_ref, wo_ref, x_ref, w2_ref, wr_ref, h2_ref, logits_ref):
    a = attn_ref[...]
    proj = jnp.concatenate(
        [jnp.dot(a[:, g * (H * HD):(g + 1) * (H * HD)], wo_ref[g],
                 preferred_element_type=jnp.float32) for g in range(G)],
        axis=-1)
    x2 = x_ref[...] + proj
    var = jnp.mean(x2 * x2, axis=-1, keepdims=True)
    h2 = w2_ref[...][None, :] * (x2 * jax.lax.rsqrt(var + EPS))
    h2_ref[...] = h2
    logits_ref[...] = jnp.dot(h2, wr_ref[...], preferred_element_type=jnp.float32)


def _k3(attn, wo, x2d, norm2_w, wr):
    return pl.pallas_call(
        _k3_body,
        grid=(NT,),
        in_specs=[
            pl.BlockSpec((TT, D), lambda t: (t, 0)),
            pl.BlockSpec((G, H * HD, H * HD), lambda t: (0, 0, 0)),
            pl.BlockSpec((TT, D), lambda t: (t, 0)),
            pl.BlockSpec((D,), lambda t: (0,)),
            pl.BlockSpec((D, E), lambda t: (0, 0)),
        ],
        out_specs=[
            pl.BlockSpec((TT, D), lambda t: (t, 0)),
            pl.BlockSpec((TT, E), lambda t: (t, 0)),
        ],
        out_shape=[
            jax.ShapeDtypeStruct((S, D), jnp.float32),
            jax.ShapeDtypeStruct((S, E), jnp.float32),
        ],
    )(attn, wo, x2d, norm2_w, wr)


CH = 512


def _k4_body(logits_ref, d1_ref, d2_ref, g1_ref, g2_ref, te_ref, lb_ref):
    logits = logits_ref[...]
    m = jnp.max(logits, axis=-1, keepdims=True)
    ex = jnp.exp(logits - m)
    probs = ex / jnp.sum(ex, axis=-1, keepdims=True)
    lanes = jax.lax.broadcasted_iota(jnp.int32, (S, E), 1)
    m1 = jnp.max(probs, axis=-1, keepdims=True)
    i1 = jnp.min(jnp.where(probs == m1, lanes, E), axis=-1, keepdims=True)
    sel1 = lanes == i1
    p2 = jnp.where(sel1, -1.0, probs)
    m2 = jnp.max(p2, axis=-1, keepdims=True)
    i2 = jnp.min(jnp.where(p2 == m2, lanes, E), axis=-1, keepdims=True)
    sel2 = lanes == i2
    denom = m1 + m2
    g1_ref[...] = m1 / denom
    g2_ref[...] = m2 / denom

    oh1 = sel1.astype(jnp.float32)
    oh2 = sel2.astype(jnp.float32)
    f = jnp.sum(oh1 + oh2, axis=0) / S
    P = jnp.sum(probs, axis=0) / S
    lb_ref[0, 0] = (E / K) * jnp.sum(f * P)

    r = jax.lax.broadcasted_iota(jnp.int32, (CH, CH), 0)
    c = jax.lax.broadcasted_iota(jnp.int32, (CH, CH), 1)
    tri = jnp.where(r > c, 1.0, 0.0)
    carry = jnp.zeros((1, E), jnp.float32)
    ranks = []
    for oh in (oh1, oh2):
        parts = []
        for ch in range(S // CH):
            blk = oh[ch * CH:(ch + 1) * CH, :]
            parts.append(jnp.dot(tri, blk, preferred_element_type=jnp.float32)
                         + carry)
            carry = carry + jnp.sum(blk, axis=0, keepdims=True)
        ranks.append(jnp.concatenate(parts, axis=0))
    counts = carry

    blocks = jnp.floor((counts + (BM - 1)) / BM)
    ru = jax.lax.broadcasted_iota(jnp.int32, (E, E), 0)
    cu = jax.lax.broadcasted_iota(jnp.int32, (E, E), 1)
    triu = jnp.where(ru < cu, 1.0, 0.0)
    off = BM * jnp.dot(blocks, triu, preferred_element_type=jnp.float32)

    d1 = jnp.sum(oh1 * (off + ranks[0]), axis=-1, keepdims=True)
    d2 = jnp.sum(oh2 * (off + ranks[1]), axis=-1, keepdims=True)
    d1_ref[...] = d1.astype(jnp.int32)
    d2_ref[...] = d2.astype(jnp.int32)

    ti = jax.lax.broadcasted_iota(jnp.int32, (8, NTILES), 1).astype(jnp.float32)
    offc = jnp.broadcast_to(off.reshape(E, 1) / BM, (E, NTILES))
    blkc = jnp.broadcast_to(blocks.reshape(E, 1), (E, NTILES))
    ind = jnp.where(jnp.logical_and(ti >= offc, ti < offc + blkc), 1.0, 0.0)
    eid = jnp.broadcast_to(
        jax.lax.broadcasted_iota(jnp.int32, (E, 1), 0).astype(jnp.float32),
        (E, NTILES))
    any_ind = jnp.sum(ind, axis=0, keepdims=True)
    te = jnp.sum(ind * eid, axis=0, keepdims=True) + 7.0 * (1.0 - any_ind)
    te_ref[...] = jnp.concatenate([te, any_ind], axis=0).astype(jnp.int32)


def _k4(logits):
    return pl.pallas_call(
        _k4_body,
        in_specs=[pl.BlockSpec((S, E), lambda: (0, 0))],
        out_specs=[
            pl.BlockSpec((S, 1), lambda: (0, 0)),
            pl.BlockSpec((S, 1), lambda: (0, 0)),
            pl.BlockSpec((S, 1), lambda: (0, 0)),
            pl.BlockSpec((S, 1), lambda: (0, 0)),
            pl.BlockSpec((2, NTILES), lambda: (0, 0)),
            pl.BlockSpec(memory_space=pltpu.SMEM),
        ],
        out_shape=[
            jax.ShapeDtypeStruct((S, 1), jnp.int32),
            jax.ShapeDtypeStruct((S, 1), jnp.int32),
            jax.ShapeDtypeStruct((S, 1), jnp.float32),
            jax.ShapeDtypeStruct((S, 1), jnp.float32),
            jax.ShapeDtypeStruct((2, NTILES), jnp.int32),
            jax.ShapeDtypeStruct((1, 1), jnp.float32),
        ],
    )(logits)


def _gelu(x):
    c = np.sqrt(2.0 / np.pi).astype(np.float32)
    return 0.5 * x * (1.0 + jnp.tanh(c * (x + 0.044715 * x * x * x)))


def _k5_body(te_ref, xs_ref, w1_ref, w2_ref, ys_ref):
    i = pl.program_id(0)

    @pl.when(te_ref[1, i] != 0)
    def _():
        x = xs_ref[...].astype(jnp.bfloat16)
        h = jnp.dot(x, w1_ref[0].astype(jnp.bfloat16),
                    preferred_element_type=jnp.float32)
        g = _gelu(h.astype(jnp.bfloat16)).astype(jnp.bfloat16)
        ys_ref[...] = jnp.dot(g, w2_ref[0].astype(jnp.bfloat16),
                              preferred_element_type=jnp.float32)


def _k5(xs, w1, w2, te):
    grid_spec = pltpu.PrefetchScalarGridSpec(
        num_scalar_prefetch=1,
        grid=(NTILES,),
        in_specs=[
            pl.BlockSpec((BM, D), lambda i, te: (i, 0)),
            pl.BlockSpec((1, D, DFF), lambda i, te: (te[0, i], 0, 0)),
            pl.BlockSpec((1, DFF, D), lambda i, te: (te[0, i], 0, 0)),
        ],
        out_specs=pl.BlockSpec((BM, D), lambda i, te: (i, 0)),
    )
    return pl.pallas_call(
        _k5_body,
        grid_spec=grid_spec,
        out_shape=jax.ShapeDtypeStruct((GPAD, D), jnp.float32),
    )(te, xs, w1, w2)


def _sc_wid():
    return lax.axis_index("s") * 2 + lax.axis_index("c")


def _scatter_body(h2_hbm, d1_hbm, d2_hbm, xs_hbm, rows_v, idx1_v, idx2_v, sem):
    base = _sc_wid() * CPW
    pltpu.sync_copy(h2_hbm.at[pl.ds(base, CPW)], rows_v)
    pltpu.sync_copy(d1_hbm.at[pl.ds(base, CPW)], idx1_v)
    pltpu.sync_copy(d2_hbm.at[pl.ds(base, CPW)], idx2_v)
    pltpu.async_copy(rows_v, xs_hbm.at[idx1_v], sem).wait()
    pltpu.async_copy(rows_v, xs_hbm.at[idx2_v], sem).wait()


def _sc_scatter(h2, d1, d2):
    mesh = plsc.VectorSubcoreMesh(core_axis_name="c", subcore_axis_name="s")
    kfn = functools.partial(
        pl.kernel, mesh=mesh,
        out_type=jax.ShapeDtypeStruct((GPAD, D), jnp.float32),
        scratch_types=[
            pltpu.VMEM((CPW, D), jnp.float32),
            pltpu.VMEM((CPW,), jnp.int32),
            pltpu.VMEM((CPW,), jnp.int32),
            pltpu.SemaphoreType.DMA,
        ],
    )(_scatter_body)
    return kfn(h2, d1, d2)


def _combine_body(ys_hbm, d1_hbm, d2_hbm, y1_hbm, y2_hbm,
                  buf1_v, buf2_v, idx1_v, idx2_v, sem1, sem2):
    base = _sc_wid() * CPW
    pltpu.sync_copy(d1_hbm.at[pl.ds(base, CPW)], idx1_v)
    pltpu.sync_copy(d2_hbm.at[pl.ds(base, CPW)], idx2_v)
    c1 = pltpu.async_copy(ys_hbm.at[idx1_v], buf1_v, sem1)
    c2 = pltpu.async_copy(ys_hbm.at[idx2_v], buf2_v, sem2)
    c1.wait()
    pltpu.sync_copy(buf1_v, y1_hbm.at[pl.ds(base, CPW)])
    c2.wait()
    pltpu.sync_copy(buf2_v, y2_hbm.at[pl.ds(base, CPW)])


def _sc_combine(ys, d1, d2):
    mesh = plsc.VectorSubcoreMesh(core_axis_name="c", subcore_axis_name="s")
    kfn = functools.partial(
        pl.kernel, mesh=mesh,
        out_type=[
            jax.ShapeDtypeStruct((S, D), jnp.float32),
            jax.ShapeDtypeStruct((S, D), jnp.float32),
        ],
        scratch_types=[
            pltpu.VMEM((CPW, D), jnp.float32),
            pltpu.VMEM((CPW, D), jnp.float32),
            pltpu.VMEM((CPW,), jnp.int32),
            pltpu.VMEM((CPW,), jnp.int32),
            pltpu.SemaphoreType.DMA,
            pltpu.SemaphoreType.DMA,
        ],
    )(_combine_body)
    return kfn(ys, d1, d2)


def _k6_body(y1_ref, y2_ref, g1_ref, g2_ref, out_ref):
    out_ref[...] = g1_ref[...] * y1_ref[...] + g2_ref[...] * y2_ref[...]


def _k6(y1, y2, g1, g2):
    return pl.pallas_call(
        _k6_body,
        grid=(NT,),
        in_specs=[
            pl.BlockSpec((TT, D), lambda t: (t, 0)),
            pl.BlockSpec((TT, D), lambda t: (t, 0)),
            pl.BlockSpec((TT, 1), lambda t: (t, 0)),
            pl.BlockSpec((TT, 1), lambda t: (t, 0)),
        ],
        out_specs=pl.BlockSpec((TT, D), lambda t: (t, 0)),
        out_shape=jax.ShapeDtypeStruct((S, D), jnp.float32),
    )(y1, y2, g1, g2)


def kernel(x, norm1_w, norm2_w, Wq, Wk, Wv, Wo, Wr, W1, W2):
    x2d = x.reshape(S, D)
    wq_all = Wq.transpose(1, 0, 2).reshape(D, D)
    wk_all = Wk.transpose(1, 0, 2).reshape(D, G * HD)
    wv_all = Wv.transpose(1, 0, 2).reshape(D, G * HD)
    wqkv = jnp.concatenate([wq_all, wk_all, wv_all], axis=1)

    half = HD // 2
    freqs = 1.0 / (10000.0 ** (jnp.arange(half, dtype=jnp.float32) / half))
    ang = jnp.arange(S, dtype=jnp.float32)[:, None] * freqs
    cos, sin = jnp.cos(ang), jnp.sin(ang)

    q, k, v = _k1(x2d, norm1_w, wqkv, cos, sin)
    q3 = q.reshape(S, G, H, HD).transpose(1, 2, 0, 3)
    kt = k.reshape(S, G, HD).transpose(1, 2, 0)
    v3 = v.reshape(S, G, HD).transpose(1, 0, 2)
    attn = _k2(q3, kt, v3)
    attn2d = attn.transpose(2, 0, 1, 3).reshape(S, NH * HD)
    h2, logits = _k3(attn2d, Wo, x2d, norm2_w, Wr)
    d1, d2, g1, g2, te, lb = _k4(logits)
    d1f, d2f = d1.reshape(S), d2.reshape(S)
    xs = _sc_scatter(h2, d1f, d2f)
    ys = _k5(xs, W1, W2, te)
    y1, y2 = _sc_combine(ys, d1f, d2f)
    out = _k6(y1, y2, g1, g2)
    return out.reshape(B, S, D), lb[0, 0]
